# Initial kernel scaffold; baseline (speedup 1.0000x reference)
import functools

import jax
import jax.numpy as jnp
from jax import lax
from jax.experimental import pallas as pl
from jax.experimental.pallas import tpu as pltpu
from jax.experimental.pallas import tpu_sc as plsc

NC = 2
NS = 16
LANES = 16
CW = 128

N = 10000
IN = 128
N_PAD = 10240
ROWS_PER_TEC = N_PAD // NS
E = 320000
CHUNKS = 2528
E_PAD = CHUNKS * CW
CH1 = CHUNKS // NS
CH2 = CHUNKS // (NC * NS)
FG = IN // LANES

_mesh = functools.partial(
    plsc.VectorSubcoreMesh, core_axis_name="c", subcore_axis_name="s")


def _rsqrt16(d):
    i = plsc.bitcast(d, jnp.int32)
    i = jnp.int32(0x5F3759DF) - lax.shift_right_logical(i, 1)
    y = plsc.bitcast(i, jnp.float32)
    half = d * jnp.float32(0.5)
    for _ in range(3):
        y = y * (jnp.float32(1.5) - half * y * y)
    return y


def _norm_tec(row_hbm, col_hbm, w_hbm, wh_hbm, r_v, c_v, w_v, dis_v, zb_v,
              deg_s):
    c = lax.axis_index("c")
    s = lax.axis_index("s")
    base = s * CH1
    pltpu.sync_copy(row_hbm.at[pl.ds(base, CH1)], r_v)
    pltpu.sync_copy(col_hbm.at[pl.ds(base, CH1)], c_v)
    pltpu.sync_copy(w_hbm.at[pl.ds(base, CH1)], w_v)

    def zz(i, _):
        zb_v[pl.ds(i * LANES, LANES)] = jnp.zeros((LANES,), jnp.float32)
        return 0
    lax.fori_loop(0, ROWS_PER_TEC // LANES, zz, 0)
    pltpu.sync_copy(zb_v, deg_s.at[pl.ds(s * ROWS_PER_TEC, ROWS_PER_TEC)])

    def mask_chunk(j, _):
        def grp(g, _):
            sl = pl.ds(g * LANES, LANES)
            rr = r_v[j, sl]
            cc = c_v[j, sl]
            w_v[j, sl] = jnp.where(rr == cc, jnp.float32(0.0), w_v[j, sl])
            return 0
        lax.fori_loop(0, FG, grp, 0)
        return 0
    lax.fori_loop(0, CH1, mask_chunk, 0)

    plsc.subcore_barrier()

    def deg_chunk(j, _):
        pltpu.sync_copy(w_v.at[j], deg_s.at[r_v.at[j]], add=True)
        return 0
    lax.fori_loop(0, CH1, deg_chunk, 0)

    plsc.subcore_barrier()

    pltpu.sync_copy(deg_s, dis_v)

    def dis_grp(i, _):
        sl = pl.ds(i * LANES, LANES)
        d = dis_v[sl]
        y = _rsqrt16(d)
        dis_v[sl] = jnp.where(d > jnp.float32(0.0), y, jnp.float32(0.0))
        return 0
    lax.fori_loop(0, N_PAD // LANES, dis_grp, 0)

    def wh_chunk(j, _):
        def grp(g, _):
            sl = pl.ds(g * LANES, LANES)
            rr = r_v[j, sl]
            cc = c_v[j, sl]
            dr = plsc.load_gather(dis_v, [rr])
            dc = plsc.load_gather(dis_v, [cc])
            w_v[j, sl] = -(dr * w_v[j, sl] * dc)
            return 0
        lax.fori_loop(0, FG, grp, 0)
        return 0
    lo = c * CH2
    lax.fori_loop(lo, lo + CH2, wh_chunk, 0)
    pltpu.sync_copy(w_v.at[pl.ds(lo, CH2)], wh_hbm.at[pl.ds(base + lo, CH2)])


@functools.partial(
    pl.kernel,
    out_type=jax.ShapeDtypeStruct((CHUNKS, CW), jnp.float32),
    mesh=_mesh(),
    scratch_types=[
        pltpu.VMEM((CH1, CW), jnp.int32),
        pltpu.VMEM((CH1, CW), jnp.int32),
        pltpu.VMEM((CH1, CW), jnp.float32),
        pltpu.VMEM((N_PAD,), jnp.float32),
        pltpu.VMEM((ROWS_PER_TEC,), jnp.float32),
        pltpu.VMEM_SHARED((N_PAD,), jnp.float32),
    ],
)
def _norm_call(row_hbm, col_hbm, w_hbm, wh_hbm, r_v, c_v, w_v, dis_v, zb_v,
               deg_s):
    _norm_tec(row_hbm, col_hbm, w_hbm, wh_hbm, r_v, c_v, w_v, dis_v, zb_v,
              deg_s)


def _prop_tec(scale, src_hbm, row_hbm, col_hbm, wh_hbm, out_hbm,
              r_v, c_v, w_v, rows_v, zb_v, acc_s):
    c = lax.axis_index("c")
    s = lax.axis_index("s")
    wid = c * NS + s
    base = wid * CH2
    pltpu.sync_copy(row_hbm.at[pl.ds(base, CH2)], r_v)
    pltpu.sync_copy(col_hbm.at[pl.ds(base, CH2)], c_v)
    pltpu.sync_copy(wh_hbm.at[pl.ds(base, CH2)], w_v)

    def zrow(r, _):
        def zg(g, _):
            zb_v[r, pl.ds(g * LANES, LANES)] = jnp.zeros((LANES,), jnp.float32)
            return 0
        lax.fori_loop(0, FG, zg, 0)
        return 0
    lax.fori_loop(0, CW, zrow, 0)
    for i in range(ROWS_PER_TEC // CW):
        pltpu.sync_copy(zb_v, acc_s.at[pl.ds(s * ROWS_PER_TEC + i * CW, CW)])
    plsc.subcore_barrier()

    sscale = jnp.float32(scale)

    def chunk(j, _):
        pltpu.sync_copy(src_hbm.at[r_v.at[j]], rows_v)

        def edge(e, _):
            bb = jnp.full((LANES,), w_v[j, e] * sscale, jnp.float32)
            for f in range(FG):
                sl = pl.ds(f * LANES, LANES)
                rows_v[e, sl] = rows_v[e, sl] * bb
            return 0
        lax.fori_loop(0, CW, edge, 0)

        pltpu.sync_copy(rows_v, acc_s.at[c_v.at[j]], add=True)
        return 0
    lax.fori_loop(0, CH2, chunk, 0)

    plsc.subcore_barrier()
    pltpu.sync_copy(acc_s.at[pl.ds(s * ROWS_PER_TEC, ROWS_PER_TEC)],
                    out_hbm.at[c, pl.ds(s * ROWS_PER_TEC, ROWS_PER_TEC)])


def _make_prop(scale):
    @functools.partial(
        pl.kernel,
        out_type=jax.ShapeDtypeStruct((NC, N_PAD, IN), jnp.float32),
        mesh=_mesh(),
        scratch_types=[
            pltpu.VMEM((CH2, CW), jnp.int32),
            pltpu.VMEM((CH2, CW), jnp.int32),
            pltpu.VMEM((CH2, CW), jnp.float32),
            pltpu.VMEM((CW, IN), jnp.float32),
            pltpu.VMEM((CW, IN), jnp.float32),
            pltpu.VMEM_SHARED((N_PAD, IN), jnp.float32),
        ],
    )
    def prop(src_hbm, row_hbm, col_hbm, wh_hbm, out_hbm,
             r_v, c_v, w_v, rows_v, zb_v, acc_s):
        _prop_tec(scale, src_hbm, row_hbm, col_hbm, wh_hbm, out_hbm,
                  r_v, c_v, w_v, rows_v, zb_v, acc_s)
    return prop


_prop1 = _make_prop(1.0)
_prop2 = _make_prop(2.0)

_RB = 1024


def _comb1_body(p_ref, o_ref):
    o_ref[...] = p_ref[0] + p_ref[1]


def _comb2_body(p_ref, z_ref, o_ref):
    o_ref[...] = p_ref[0] + p_ref[1] - z_ref[...]


_comb1 = pl.pallas_call(
    _comb1_body,
    grid=(N_PAD // _RB,),
    in_specs=[pl.BlockSpec((NC, _RB, IN), lambda i: (0, i, 0))],
    out_specs=pl.BlockSpec((_RB, IN), lambda i: (i, 0)),
    out_shape=jax.ShapeDtypeStruct((N_PAD, IN), jnp.float32),
)

_comb2 = pl.pallas_call(
    _comb2_body,
    grid=(N_PAD // _RB,),
    in_specs=[pl.BlockSpec((NC, _RB, IN), lambda i: (0, i, 0)),
              pl.BlockSpec((_RB, IN), lambda i: (i, 0))],
    out_specs=pl.BlockSpec((_RB, IN), lambda i: (i, 0)),
    out_shape=jax.ShapeDtypeStruct((N_PAD, IN), jnp.float32),
)

_MB = 1000
OUT3 = 300


def _mm_body(x_ref, t1_ref, t2_ref, p3_ref, w_ref, b_ref, o_ref):
    t1 = t1_ref[...]
    t3 = p3_ref[0] + p3_ref[1] - t1
    acc = jnp.dot(x_ref[...], w_ref[0], preferred_element_type=jnp.float32)
    acc = acc + jnp.dot(t1, w_ref[1], preferred_element_type=jnp.float32)
    acc = acc + jnp.dot(t2_ref[...], w_ref[2], preferred_element_type=jnp.float32)
    acc = acc + jnp.dot(t3, w_ref[3], preferred_element_type=jnp.float32)
    o_ref[...] = acc + b_ref[0:1, :]


_mm = pl.pallas_call(
    _mm_body,
    grid=(N // _MB,),
    in_specs=[pl.BlockSpec((_MB, IN), lambda i: (i, 0)),
              pl.BlockSpec((_MB, IN), lambda i: (i, 0)),
              pl.BlockSpec((_MB, IN), lambda i: (i, 0)),
              pl.BlockSpec((NC, _MB, IN), lambda i: (0, i, 0)),
              pl.BlockSpec((4, IN, OUT3), lambda i: (0, 0, 0)),
              pl.BlockSpec((8, OUT3), lambda i: (0, 0))],
    out_specs=pl.BlockSpec((_MB, OUT3), lambda i: (i, 0)),
    out_shape=jax.ShapeDtypeStruct((N, OUT3), jnp.float32),
)


def kernel(x, edge_index, edge_weight, W1, b1, W2, b2, W3, b3):
    row = edge_index[0]
    col = edge_index[1]
    e = row.shape[0]
    pad = E_PAD - e
    row2 = jnp.pad(row, (0, pad)).reshape(CHUNKS, CW)
    col2 = jnp.pad(col, (0, pad)).reshape(CHUNKS, CW)
    w2 = jnp.pad(edge_weight, (0, pad)).reshape(CHUNKS, CW)
    x_p = jnp.pad(x, ((0, N_PAD - x.shape[0]), (0, 0)))

    wh = _norm_call(row2, col2, w2)

    p1 = _prop1(x_p, row2, col2, wh)
    tx1 = _comb1(p1)
    p2 = _prop2(tx1, row2, col2, wh)
    tx2 = _comb2(p2, x_p)
    p3 = _prop2(tx2, row2, col2, wh)

    zero = jnp.zeros((IN, 100), jnp.float32)
    wc = jnp.stack([
        jnp.concatenate([W1[0], W2[0], W3[0]], axis=1),
        jnp.concatenate([W1[1], W2[1], W3[1]], axis=1),
        jnp.concatenate([zero, W2[2], W3[2]], axis=1),
        jnp.concatenate([zero, zero, W3[3]], axis=1),
    ])
    bc = jnp.tile(jnp.concatenate([b1, b2, b3])[None, :], (8, 1))

    return _mm(x_p, tx1, tx2, p3, wc, bc)


# speedup vs baseline: 4.8610x; 4.8610x over previous
import functools

import jax
import jax.numpy as jnp
from jax import lax
from jax.experimental import pallas as pl
from jax.experimental.pallas import tpu as pltpu
from jax.experimental.pallas import tpu_sc as plsc

NC = 2
NS = 16
LANES = 16
CW = 128

N = 10000
IN = 128
N_PAD = 10240
ROWS_PER_TEC = N_PAD // NS
E = 320000
CHUNKS = 2560
E_PAD = CHUNKS * CW
CH1 = CHUNKS // NS
CH2 = CHUNKS // (NC * NS)
FG = IN // LANES

_mesh = functools.partial(
    plsc.VectorSubcoreMesh, core_axis_name="c", subcore_axis_name="s")


def _rsqrt16(d):
    i = lax.bitcast_convert_type(d, jnp.int32)
    i = jnp.int32(0x5F3759DF) - lax.shift_right_logical(i, 1)
    y = lax.bitcast_convert_type(i, jnp.float32)
    half = d * jnp.float32(0.5)
    for _ in range(3):
        y = y * (jnp.float32(1.5) - half * y * y)
    return y


def _norm_tec(row_hbm, col_hbm, w_hbm, wh_hbm, r_v, c_v, w_v, dis_v, zb_v,
              deg_s):
    c = lax.axis_index("c")
    s = lax.axis_index("s")
    base = s * CH1
    pltpu.sync_copy(row_hbm.at[pl.ds(base, CH1)], r_v)
    pltpu.sync_copy(col_hbm.at[pl.ds(base, CH1)], c_v)
    pltpu.sync_copy(w_hbm.at[pl.ds(base, CH1)], w_v)

    def zz(i, _):
        zb_v[pl.ds(i * LANES, LANES)] = jnp.zeros((LANES,), jnp.float32)
        return 0
    lax.fori_loop(0, ROWS_PER_TEC // LANES, zz, 0)
    pltpu.sync_copy(zb_v, deg_s.at[pl.ds(s * ROWS_PER_TEC, ROWS_PER_TEC)])

    def mask_chunk(j, _):
        def grp(g, _):
            sl = pl.ds(g * LANES, LANES)
            rr = r_v[j, sl]
            cc = c_v[j, sl]
            w_v[j, sl] = jnp.where(rr == cc, jnp.float32(0.0), w_v[j, sl])
            return 0
        lax.fori_loop(0, FG, grp, 0)
        return 0
    lax.fori_loop(0, CH1, mask_chunk, 0)

    plsc.subcore_barrier()

    def deg_chunk(j, _):
        pltpu.sync_copy(w_v.at[j], deg_s.at[r_v.at[j]], add=True)
        return 0
    lax.fori_loop(0, CH1, deg_chunk, 0)

    plsc.subcore_barrier()

    pltpu.sync_copy(deg_s, dis_v)

    def dis_grp(i, _):
        sl = pl.ds(i * LANES, LANES)
        d = dis_v[sl]
        y = _rsqrt16(d)
        dis_v[sl] = jnp.where(d > jnp.float32(0.0), y, jnp.float32(0.0))
        return 0
    lax.fori_loop(0, N_PAD // LANES, dis_grp, 0)

    def wh_chunk(j, _):
        def grp(g, _):
            sl = pl.ds(g * LANES, LANES)
            rr = r_v[j, sl]
            cc = c_v[j, sl]
            dr = plsc.load_gather(dis_v, [rr])
            dc = plsc.load_gather(dis_v, [cc])
            w_v[j, sl] = -(dr * w_v[j, sl] * dc)
            return 0
        lax.fori_loop(0, FG, grp, 0)
        return 0
    lo = c * CH2
    lax.fori_loop(lo, lo + CH2, wh_chunk, 0)
    pltpu.sync_copy(w_v.at[pl.ds(lo, CH2)], wh_hbm.at[pl.ds(base + lo, CH2)])


@functools.partial(
    pl.kernel,
    out_type=jax.ShapeDtypeStruct((CHUNKS, CW), jnp.float32),
    mesh=_mesh(),
    compiler_params=pltpu.CompilerParams(needs_layout_passes=False),
    scratch_types=[
        pltpu.VMEM((CH1, CW), jnp.int32),
        pltpu.VMEM((CH1, CW), jnp.int32),
        pltpu.VMEM((CH1, CW), jnp.float32),
        pltpu.VMEM((N_PAD,), jnp.float32),
        pltpu.VMEM((ROWS_PER_TEC,), jnp.float32),
        pltpu.VMEM_SHARED((N_PAD,), jnp.float32),
    ],
)
def _norm_call(row_hbm, col_hbm, w_hbm, wh_hbm, r_v, c_v, w_v, dis_v, zb_v,
               deg_s):
    _norm_tec(row_hbm, col_hbm, w_hbm, wh_hbm, r_v, c_v, w_v, dis_v, zb_v,
              deg_s)


def _prop_tec(scale, src_hbm, row_hbm, col_hbm, wh_hbm, out_hbm,
              r_v, c_v, w_v, rows_v, acc_s):
    c = lax.axis_index("c")
    s = lax.axis_index("s")
    wid = c * NS + s
    base = wid * CH2
    pltpu.sync_copy(row_hbm.at[pl.ds(base, CH2)], r_v)
    pltpu.sync_copy(col_hbm.at[pl.ds(base, CH2)], c_v)
    pltpu.sync_copy(wh_hbm.at[pl.ds(base, CH2)], w_v)

    def zrow(r, _):
        def zg(g, _):
            rows_v[r, pl.ds(g * LANES, LANES)] = jnp.zeros((LANES,), jnp.float32)
            return 0
        lax.fori_loop(0, FG, zg, 0)
        return 0
    lax.fori_loop(0, CW, zrow, 0)
    for i in range(ROWS_PER_TEC // CW):
        pltpu.sync_copy(rows_v, acc_s.at[pl.ds(s * ROWS_PER_TEC + i * CW, CW)])
    plsc.subcore_barrier()

    sscale = jnp.float32(scale)

    def chunk(j, _):
        pltpu.sync_copy(src_hbm.at[r_v.at[j]], rows_v)

        jj = jnp.full((LANES,), j, jnp.int32)

        def edge(e, _):
            ee = jnp.full((LANES,), e, jnp.int32)
            bb = plsc.load_gather(w_v, [jj, ee]) * sscale
            for f in range(FG):
                sl = pl.ds(f * LANES, LANES)
                rows_v[e, sl] = rows_v[e, sl] * bb
            return 0
        lax.fori_loop(0, CW, edge, 0)

        pltpu.sync_copy(rows_v, acc_s.at[c_v.at[j]], add=True)
        return 0
    lax.fori_loop(0, CH2, chunk, 0)

    plsc.subcore_barrier()
    pltpu.sync_copy(acc_s.at[pl.ds(s * ROWS_PER_TEC, ROWS_PER_TEC)],
                    out_hbm.at[c, pl.ds(s * ROWS_PER_TEC, ROWS_PER_TEC)])


def _make_prop(scale):
    @functools.partial(
        pl.kernel,
        out_type=jax.ShapeDtypeStruct((NC, N_PAD, IN), jnp.float32),
        mesh=_mesh(),
        compiler_params=pltpu.CompilerParams(needs_layout_passes=False),
        scratch_types=[
            pltpu.VMEM((CH2, CW), jnp.int32),
            pltpu.VMEM((CH2, CW), jnp.int32),
            pltpu.VMEM((CH2, CW), jnp.float32),
            pltpu.VMEM((CW, IN), jnp.float32),
            pltpu.VMEM_SHARED((N_PAD, IN), jnp.float32),
        ],
    )
    def prop(src_hbm, row_hbm, col_hbm, wh_hbm, out_hbm,
             r_v, c_v, w_v, rows_v, acc_s):
        _prop_tec(scale, src_hbm, row_hbm, col_hbm, wh_hbm, out_hbm,
                  r_v, c_v, w_v, rows_v, acc_s)
    return prop


_prop1 = _make_prop(1.0)
_prop2 = _make_prop(2.0)

_RB = 1024


def _comb1_body(p_ref, o_ref):
    o_ref[...] = p_ref[0] + p_ref[1]


def _comb2_body(p_ref, z_ref, o_ref):
    o_ref[...] = p_ref[0] + p_ref[1] - z_ref[...]


_comb1 = pl.pallas_call(
    _comb1_body,
    grid=(N_PAD // _RB,),
    in_specs=[pl.BlockSpec((NC, _RB, IN), lambda i: (0, i, 0))],
    out_specs=pl.BlockSpec((_RB, IN), lambda i: (i, 0)),
    out_shape=jax.ShapeDtypeStruct((N_PAD, IN), jnp.float32),
)

_comb2 = pl.pallas_call(
    _comb2_body,
    grid=(N_PAD // _RB,),
    in_specs=[pl.BlockSpec((NC, _RB, IN), lambda i: (0, i, 0)),
              pl.BlockSpec((_RB, IN), lambda i: (i, 0))],
    out_specs=pl.BlockSpec((_RB, IN), lambda i: (i, 0)),
    out_shape=jax.ShapeDtypeStruct((N_PAD, IN), jnp.float32),
)

_MB = 1000
OUT3 = 300


def _mm_body(x_ref, t1_ref, t2_ref, p3_ref, w_ref, b_ref, o_ref):
    t1 = t1_ref[...]
    t3 = p3_ref[0] + p3_ref[1] - t1
    acc = jnp.dot(x_ref[...], w_ref[0], preferred_element_type=jnp.float32)
    acc = acc + jnp.dot(t1, w_ref[1], preferred_element_type=jnp.float32)
    acc = acc + jnp.dot(t2_ref[...], w_ref[2], preferred_element_type=jnp.float32)
    acc = acc + jnp.dot(t3, w_ref[3], preferred_element_type=jnp.float32)
    o_ref[...] = acc + b_ref[0:1, :]


_mm = pl.pallas_call(
    _mm_body,
    grid=(N // _MB,),
    in_specs=[pl.BlockSpec((_MB, IN), lambda i: (i, 0)),
              pl.BlockSpec((_MB, IN), lambda i: (i, 0)),
              pl.BlockSpec((_MB, IN), lambda i: (i, 0)),
              pl.BlockSpec((NC, _MB, IN), lambda i: (0, i, 0)),
              pl.BlockSpec((4, IN, OUT3), lambda i: (0, 0, 0)),
              pl.BlockSpec((8, OUT3), lambda i: (0, 0))],
    out_specs=pl.BlockSpec((_MB, OUT3), lambda i: (i, 0)),
    out_shape=jax.ShapeDtypeStruct((N, OUT3), jnp.float32),
)


def kernel(x, edge_index, edge_weight, W1, b1, W2, b2, W3, b3):
    row = edge_index[0]
    col = edge_index[1]
    e = row.shape[0]
    pad = E_PAD - e
    row2 = jnp.pad(row, (0, pad)).reshape(CHUNKS, CW)
    col2 = jnp.pad(col, (0, pad)).reshape(CHUNKS, CW)
    w2 = jnp.pad(edge_weight, (0, pad)).reshape(CHUNKS, CW)
    x_p = jnp.pad(x, ((0, N_PAD - x.shape[0]), (0, 0)))

    wh = _norm_call(row2, col2, w2)

    p1 = _prop1(x_p, row2, col2, wh)
    tx1 = _comb1(p1)
    p2 = _prop2(tx1, row2, col2, wh)
    tx2 = _comb2(p2, x_p)
    p3 = _prop2(tx2, row2, col2, wh)

    zero = jnp.zeros((IN, 100), jnp.float32)
    wc = jnp.stack([
        jnp.concatenate([W1[0], W2[0], W3[0]], axis=1),
        jnp.concatenate([W1[1], W2[1], W3[1]], axis=1),
        jnp.concatenate([zero, W2[2], W3[2]], axis=1),
        jnp.concatenate([zero, zero, W3[3]], axis=1),
    ])
    bc = jnp.tile(jnp.concatenate([b1, b2, b3])[None, :], (8, 1))

    return _mm(x_p, tx1, tx2, p3, wc, bc)


# speedup vs baseline: 5.4290x; 1.1168x over previous
import functools

import jax
import jax.numpy as jnp
from jax import lax
from jax.experimental import pallas as pl
from jax.experimental.pallas import tpu as pltpu
from jax.experimental.pallas import tpu_sc as plsc

NC = 2
NS = 16
LANES = 16
CW = 128

N = 10000
IN = 128
N_PAD = 10240
ROWS_PER_TEC = N_PAD // NS
E = 320000
CHUNKS = 2560
E_PAD = CHUNKS * CW
CH1 = CHUNKS // NS
CH2 = CHUNKS // (NC * NS)
FG = IN // LANES

_mesh = functools.partial(
    plsc.VectorSubcoreMesh, core_axis_name="c", subcore_axis_name="s")


def _rsqrt16(d):
    i = lax.bitcast_convert_type(d, jnp.int32)
    i = jnp.int32(0x5F3759DF) - lax.shift_right_logical(i, 1)
    y = lax.bitcast_convert_type(i, jnp.float32)
    half = d * jnp.float32(0.5)
    for _ in range(3):
        y = y * (jnp.float32(1.5) - half * y * y)
    return y


def _norm_tec(row_hbm, col_hbm, w_hbm, wh_hbm, r_v, c_v, w_v, dis_v, zb_v,
              deg_s, sdma):
    c = lax.axis_index("c")
    s = lax.axis_index("s")
    base = s * CH1
    pltpu.sync_copy(row_hbm.at[pl.ds(base, CH1)], r_v)
    pltpu.sync_copy(col_hbm.at[pl.ds(base, CH1)], c_v)
    pltpu.sync_copy(w_hbm.at[pl.ds(base, CH1)], w_v)

    def zz(i, _):
        zb_v[pl.ds(i * LANES, LANES)] = jnp.zeros((LANES,), jnp.float32)
        return 0
    lax.fori_loop(0, ROWS_PER_TEC // LANES, zz, 0)
    pltpu.sync_copy(zb_v, deg_s.at[pl.ds(s * ROWS_PER_TEC, ROWS_PER_TEC)])

    def mask_chunk(j, _):
        def grp(g, _):
            sl = pl.ds(g * LANES, LANES)
            rr = r_v[j, sl]
            cc = c_v[j, sl]
            w_v[j, sl] = jnp.where(rr == cc, jnp.float32(0.0), w_v[j, sl])
            return 0
        lax.fori_loop(0, FG, grp, 0)
        return 0
    lax.fori_loop(0, CH1, mask_chunk, 0)

    plsc.subcore_barrier()

    def deg_blk(b, _):
        for u in range(8):
            j = b * 8 + u
            pltpu.async_copy(w_v.at[j], deg_s.at[r_v.at[j]], sdma, add=True)
        for u in range(8):
            j = b * 8 + u
            pltpu.make_async_copy(w_v.at[j], deg_s.at[r_v.at[j]], sdma).wait()
        return 0
    lax.fori_loop(0, CH1 // 8, deg_blk, 0)

    plsc.subcore_barrier()

    pltpu.sync_copy(deg_s, dis_v)

    def dis_grp(i, _):
        sl = pl.ds(i * LANES, LANES)
        d = dis_v[sl]
        y = _rsqrt16(d)
        dis_v[sl] = jnp.where(d > jnp.float32(0.0), y, jnp.float32(0.0))
        return 0
    lax.fori_loop(0, N_PAD // LANES, dis_grp, 0)

    def wh_chunk(j, _):
        def grp(g, _):
            sl = pl.ds(g * LANES, LANES)
            rr = r_v[j, sl]
            cc = c_v[j, sl]
            dr = plsc.load_gather(dis_v, [rr])
            dc = plsc.load_gather(dis_v, [cc])
            w_v[j, sl] = -(dr * w_v[j, sl] * dc)
            return 0
        lax.fori_loop(0, FG, grp, 0)
        return 0
    lo = c * CH2
    lax.fori_loop(lo, lo + CH2, wh_chunk, 0)
    pltpu.sync_copy(w_v.at[pl.ds(lo, CH2)], wh_hbm.at[pl.ds(base + lo, CH2)])


@functools.partial(
    pl.kernel,
    out_type=jax.ShapeDtypeStruct((CHUNKS, CW), jnp.float32),
    mesh=_mesh(),
    compiler_params=pltpu.CompilerParams(needs_layout_passes=False),
    scratch_types=[
        pltpu.VMEM((CH1, CW), jnp.int32),
        pltpu.VMEM((CH1, CW), jnp.int32),
        pltpu.VMEM((CH1, CW), jnp.float32),
        pltpu.VMEM((N_PAD,), jnp.float32),
        pltpu.VMEM((ROWS_PER_TEC,), jnp.float32),
        pltpu.VMEM_SHARED((N_PAD,), jnp.float32),
        pltpu.SemaphoreType.DMA,
    ],
)
def _norm_call(row_hbm, col_hbm, w_hbm, wh_hbm, r_v, c_v, w_v, dis_v, zb_v,
               deg_s, sdma):
    _norm_tec(row_hbm, col_hbm, w_hbm, wh_hbm, r_v, c_v, w_v, dis_v, zb_v,
              deg_s, sdma)


def _prop_tec(scale, src_hbm, col_hbm, wh_hbm, row_hbm, out_hbm,
              rows_a, rows_b, col_a, col_b, wb_a, wb_b, row_a, row_b, acc_s,
              sga, sgb, ssa, ssb, sia, sib):
    c = lax.axis_index("c")
    s = lax.axis_index("s")
    wid = c * NS + s
    ebase = wid * CH2 * CW
    sscale = jnp.float32(scale)

    def zrow(r, _):
        def zg(g, _):
            rows_a[r, pl.ds(g * LANES, LANES)] = jnp.zeros((LANES,), jnp.float32)
            return 0
        lax.fori_loop(0, FG, zg, 0)
        return 0
    lax.fori_loop(0, CW, zrow, 0)
    for i in range(ROWS_PER_TEC // CW):
        pltpu.sync_copy(rows_a, acc_s.at[pl.ds(s * ROWS_PER_TEC + i * CW, CW)])
    plsc.subcore_barrier()

    def issue_idx(j, cv, wv, rv, sem):
        off = ebase + j * CW
        pltpu.async_copy(col_hbm.at[pl.ds(off, CW)], cv, sem)
        pltpu.async_copy(wh_hbm.at[pl.ds(off, CW)], wv, sem)
        pltpu.async_copy(row_hbm.at[pl.ds(off, CW)], rv, sem)

    def issue_rw(j, wv, rv, sem):
        off = ebase + j * CW
        pltpu.async_copy(wh_hbm.at[pl.ds(off, CW)], wv, sem)
        pltpu.async_copy(row_hbm.at[pl.ds(off, CW)], rv, sem)

    def issue_col(j, cv, sem):
        off = ebase + j * CW
        pltpu.async_copy(col_hbm.at[pl.ds(off, CW)], cv, sem)

    def wait_idx(cv, wv, rv, sem):
        pltpu.make_async_copy(col_hbm.at[pl.ds(0, CW)], cv, sem).wait()
        pltpu.make_async_copy(wh_hbm.at[pl.ds(0, CW)], wv, sem).wait()
        pltpu.make_async_copy(row_hbm.at[pl.ds(0, CW)], rv, sem).wait()

    def issue_gather(rv, buf, sem):
        pltpu.async_copy(src_hbm.at[rv], buf, sem)

    def wait_gather(rv, buf, sem):
        pltpu.make_async_copy(src_hbm.at[rv], buf, sem).wait()

    def issue_scatter(cv, buf, sem):
        pltpu.async_copy(buf, acc_s.at[cv], sem, add=True)

    def wait_scatter(cv, buf, sem):
        pltpu.make_async_copy(buf, acc_s.at[cv], sem).wait()

    def scale_buf(wv, buf):
        def edge(e2, _):
            for u in range(2):
                e = e2 * 2 + u
                ee = jnp.full((LANES,), e, jnp.int32)
                bb = plsc.load_gather(wv, [ee]) * sscale
                for f in range(FG):
                    sl = pl.ds(f * LANES, LANES)
                    buf[e, sl] = buf[e, sl] * bb
            return 0
        lax.fori_loop(0, CW // 2, edge, 0)

    issue_idx(0, col_a, wb_a, row_a, sia)
    issue_idx(1, col_b, wb_b, row_b, sib)
    wait_idx(col_a, wb_a, row_a, sia)
    issue_gather(row_a, rows_a, sga)
    wait_idx(col_b, wb_b, row_b, sib)
    issue_gather(row_b, rows_b, sgb)

    def pair(p, _):
        j0 = p * 2

        wait_gather(row_a, rows_a, sga)
        scale_buf(wb_a, rows_a)
        issue_scatter(col_a, rows_a, ssa)

        @pl.when(j0 + 2 < CH2)
        def _():
            issue_rw(j0 + 2, wb_a, row_a, sia)

        wait_gather(row_b, rows_b, sgb)
        scale_buf(wb_b, rows_b)
        issue_scatter(col_b, rows_b, ssb)

        @pl.when(j0 + 3 < CH2)
        def _():
            issue_rw(j0 + 3, wb_b, row_b, sib)

        wait_scatter(col_a, rows_a, ssa)

        @pl.when(j0 + 2 < CH2)
        def _():
            issue_col(j0 + 2, col_a, sia)
            wait_idx(col_a, wb_a, row_a, sia)
            issue_gather(row_a, rows_a, sga)

        wait_scatter(col_b, rows_b, ssb)

        @pl.when(j0 + 3 < CH2)
        def _():
            issue_col(j0 + 3, col_b, sib)
            wait_idx(col_b, wb_b, row_b, sib)
            issue_gather(row_b, rows_b, sgb)
        return 0
    lax.fori_loop(0, CH2 // 2, pair, 0)

    plsc.subcore_barrier()
    pltpu.sync_copy(acc_s.at[pl.ds(s * ROWS_PER_TEC, ROWS_PER_TEC)],
                    out_hbm.at[c, pl.ds(s * ROWS_PER_TEC, ROWS_PER_TEC)])


def _make_prop(scale):
    @functools.partial(
        pl.kernel,
        out_type=jax.ShapeDtypeStruct((NC, N_PAD, IN), jnp.float32),
        mesh=_mesh(),
        compiler_params=pltpu.CompilerParams(needs_layout_passes=False),
        scratch_types=[
            pltpu.VMEM((CW, IN), jnp.float32),
            pltpu.VMEM((CW, IN), jnp.float32),
            pltpu.VMEM((CW,), jnp.int32),
            pltpu.VMEM((CW,), jnp.int32),
            pltpu.VMEM((CW,), jnp.float32),
            pltpu.VMEM((CW,), jnp.float32),
            pltpu.VMEM((CW,), jnp.int32),
            pltpu.VMEM((CW,), jnp.int32),
            pltpu.VMEM_SHARED((N_PAD, IN), jnp.float32),
            pltpu.SemaphoreType.DMA,
            pltpu.SemaphoreType.DMA,
            pltpu.SemaphoreType.DMA,
            pltpu.SemaphoreType.DMA,
            pltpu.SemaphoreType.DMA,
            pltpu.SemaphoreType.DMA,
        ],
    )
    def prop(src_hbm, col_hbm, wh_hbm, row_hbm, out_hbm,
             rows_a, rows_b, col_a, col_b, wb_a, wb_b, row_a, row_b, acc_s,
             sga, sgb, ssa, ssb, sia, sib):
        _prop_tec(scale, src_hbm, col_hbm, wh_hbm, row_hbm, out_hbm,
                  rows_a, rows_b, col_a, col_b, wb_a, wb_b, row_a, row_b,
                  acc_s, sga, sgb, ssa, ssb, sia, sib)
    return prop


_prop1 = _make_prop(1.0)
_prop2 = _make_prop(2.0)

_RB = 1024


def _comb1_body(p_ref, o_ref):
    o_ref[...] = p_ref[0] + p_ref[1]


def _comb2_body(p_ref, z_ref, o_ref):
    o_ref[...] = p_ref[0] + p_ref[1] - z_ref[...]


_comb1 = pl.pallas_call(
    _comb1_body,
    grid=(N_PAD // _RB,),
    in_specs=[pl.BlockSpec((NC, _RB, IN), lambda i: (0, i, 0))],
    out_specs=pl.BlockSpec((_RB, IN), lambda i: (i, 0)),
    out_shape=jax.ShapeDtypeStruct((N_PAD, IN), jnp.float32),
)

_comb2 = pl.pallas_call(
    _comb2_body,
    grid=(N_PAD // _RB,),
    in_specs=[pl.BlockSpec((NC, _RB, IN), lambda i: (0, i, 0)),
              pl.BlockSpec((_RB, IN), lambda i: (i, 0))],
    out_specs=pl.BlockSpec((_RB, IN), lambda i: (i, 0)),
    out_shape=jax.ShapeDtypeStruct((N_PAD, IN), jnp.float32),
)

_MB = 1000
OUT3 = 300


def _mm_body(x_ref, t1_ref, t2_ref, p3_ref, w_ref, b_ref, o_ref):
    t1 = t1_ref[...]
    t3 = p3_ref[0] + p3_ref[1] - t1
    acc = jnp.dot(x_ref[...], w_ref[0], preferred_element_type=jnp.float32)
    acc = acc + jnp.dot(t1, w_ref[1], preferred_element_type=jnp.float32)
    acc = acc + jnp.dot(t2_ref[...], w_ref[2], preferred_element_type=jnp.float32)
    acc = acc + jnp.dot(t3, w_ref[3], preferred_element_type=jnp.float32)
    o_ref[...] = acc + b_ref[0:1, :]


_mm = pl.pallas_call(
    _mm_body,
    grid=(N // _MB,),
    in_specs=[pl.BlockSpec((_MB, IN), lambda i: (i, 0)),
              pl.BlockSpec((_MB, IN), lambda i: (i, 0)),
              pl.BlockSpec((_MB, IN), lambda i: (i, 0)),
              pl.BlockSpec((NC, _MB, IN), lambda i: (0, i, 0)),
              pl.BlockSpec((4, IN, OUT3), lambda i: (0, 0, 0)),
              pl.BlockSpec((8, OUT3), lambda i: (0, 0))],
    out_specs=pl.BlockSpec((_MB, OUT3), lambda i: (i, 0)),
    out_shape=jax.ShapeDtypeStruct((N, OUT3), jnp.float32),
)


def kernel(x, edge_index, edge_weight, W1, b1, W2, b2, W3, b3):
    row = edge_index[0]
    col = edge_index[1]
    e = row.shape[0]
    pad = E_PAD - e
    row2 = jnp.pad(row, (0, pad)).reshape(CHUNKS, CW)
    col2 = jnp.pad(col, (0, pad)).reshape(CHUNKS, CW)
    w2 = jnp.pad(edge_weight, (0, pad)).reshape(CHUNKS, CW)
    x_p = jnp.pad(x, ((0, N_PAD - x.shape[0]), (0, 0)))

    wh = _norm_call(row2, col2, w2)
    wh1 = wh.reshape(E_PAD)
    col1 = col2.reshape(E_PAD)
    row1 = row2.reshape(E_PAD)

    p1 = _prop1(x_p, col1, wh1, row1)
    tx1 = _comb1(p1)
    p2 = _prop2(tx1, col1, wh1, row1)
    tx2 = _comb2(p2, x_p)
    p3 = _prop2(tx2, col1, wh1, row1)

    zero = jnp.zeros((IN, 100), jnp.float32)
    wc = jnp.stack([
        jnp.concatenate([W1[0], W2[0], W3[0]], axis=1),
        jnp.concatenate([W1[1], W2[1], W3[1]], axis=1),
        jnp.concatenate([zero, W2[2], W3[2]], axis=1),
        jnp.concatenate([zero, zero, W3[3]], axis=1),
    ])
    bc = jnp.tile(jnp.concatenate([b1, b2, b3])[None, :], (8, 1))

    return _mm(x_p, tx1, tx2, p3, wc, bc)


# speedup vs baseline: 13.6461x; 2.5135x over previous
import functools

import jax
import jax.numpy as jnp
from jax import lax
from jax.experimental import pallas as pl
from jax.experimental.pallas import tpu as pltpu
from jax.experimental.pallas import tpu_sc as plsc

NC = 2
NS = 16
LANES = 16
CW = 128

N = 10000
IN = 128
N_PAD = 10240
ROWS_PER_TEC = N_PAD // NS
E = 320000
CHUNKS = 2560
E_PAD = CHUNKS * CW
CH1 = CHUNKS // NS
CH2 = CHUNKS // (NC * NS)
FG = IN // LANES

_mesh = functools.partial(
    plsc.VectorSubcoreMesh, core_axis_name="c", subcore_axis_name="s")


def _rsqrt16(d):
    i = lax.bitcast_convert_type(d, jnp.int32)
    i = jnp.int32(0x5F3759DF) - lax.shift_right_logical(i, 1)
    y = lax.bitcast_convert_type(i, jnp.float32)
    half = d * jnp.float32(0.5)
    for _ in range(3):
        y = y * (jnp.float32(1.5) - half * y * y)
    return y


def _norm_tec(row_hbm, col_hbm, w_hbm, wh_hbm, r_v, c_v, w_v, dis_v, zb_v,
              deg_s, sdma):
    c = lax.axis_index("c")
    s = lax.axis_index("s")
    base = s * CH1
    pltpu.sync_copy(row_hbm.at[pl.ds(base, CH1)], r_v)
    pltpu.sync_copy(col_hbm.at[pl.ds(base, CH1)], c_v)
    pltpu.sync_copy(w_hbm.at[pl.ds(base, CH1)], w_v)

    def zz(i, _):
        zb_v[pl.ds(i * LANES, LANES)] = jnp.zeros((LANES,), jnp.float32)
        return 0
    lax.fori_loop(0, ROWS_PER_TEC // LANES, zz, 0)
    pltpu.sync_copy(zb_v, deg_s.at[pl.ds(s * ROWS_PER_TEC, ROWS_PER_TEC)])

    def mask_chunk(j, _):
        def grp(g, _):
            sl = pl.ds(g * LANES, LANES)
            rr = r_v[j, sl]
            cc = c_v[j, sl]
            w_v[j, sl] = jnp.where(rr == cc, jnp.float32(0.0), w_v[j, sl])
            return 0
        lax.fori_loop(0, FG, grp, 0)
        return 0
    lax.fori_loop(0, CH1, mask_chunk, 0)

    plsc.subcore_barrier()

    def deg_blk(b, _):
        for u in range(8):
            j = b * 8 + u
            pltpu.async_copy(w_v.at[j], deg_s.at[r_v.at[j]], sdma, add=True)
        for u in range(8):
            j = b * 8 + u
            pltpu.make_async_copy(w_v.at[j], deg_s.at[r_v.at[j]], sdma).wait()
        return 0
    lax.fori_loop(0, CH1 // 8, deg_blk, 0)

    plsc.subcore_barrier()

    pltpu.sync_copy(deg_s, dis_v)

    def dis_grp(i, _):
        sl = pl.ds(i * LANES, LANES)
        d = dis_v[sl]
        y = _rsqrt16(d)
        dis_v[sl] = jnp.where(d > jnp.float32(0.0), y, jnp.float32(0.0))
        return 0
    lax.fori_loop(0, N_PAD // LANES, dis_grp, 0)

    def wh_chunk(j, _):
        def grp(g, _):
            sl = pl.ds(g * LANES, LANES)
            rr = r_v[j, sl]
            cc = c_v[j, sl]
            dr = plsc.load_gather(dis_v, [rr])
            dc = plsc.load_gather(dis_v, [cc])
            w_v[j, sl] = -(dr * w_v[j, sl] * dc)
            return 0
        lax.fori_loop(0, FG, grp, 0)
        return 0
    lo = c * CH2
    lax.fori_loop(lo, lo + CH2, wh_chunk, 0)
    pltpu.sync_copy(w_v.at[pl.ds(lo, CH2)], wh_hbm.at[pl.ds(base + lo, CH2)])


@functools.partial(
    pl.kernel,
    out_type=jax.ShapeDtypeStruct((CHUNKS, CW), jnp.float32),
    mesh=_mesh(),
    compiler_params=pltpu.CompilerParams(needs_layout_passes=False),
    scratch_types=[
        pltpu.VMEM((CH1, CW), jnp.int32),
        pltpu.VMEM((CH1, CW), jnp.int32),
        pltpu.VMEM((CH1, CW), jnp.float32),
        pltpu.VMEM((N_PAD,), jnp.float32),
        pltpu.VMEM((ROWS_PER_TEC,), jnp.float32),
        pltpu.VMEM_SHARED((N_PAD,), jnp.float32),
        pltpu.SemaphoreType.DMA,
    ],
)
def _norm_call(row_hbm, col_hbm, w_hbm, wh_hbm, r_v, c_v, w_v, dis_v, zb_v,
               deg_s, sdma):
    _norm_tec(row_hbm, col_hbm, w_hbm, wh_hbm, r_v, c_v, w_v, dis_v, zb_v,
              deg_s, sdma)


def _prop_tec(scale, src_hbm, col_hbm, wh_hbm, row_hbm, out_hbm,
              rows_a, rows_b, col_a, col_b, wb_a, wb_b, row_a, row_b, acc_s,
              sga, sgb, ssa, ssb, sia, sib):
    c = lax.axis_index("c")
    s = lax.axis_index("s")
    wid = c * NS + s
    sscale = jnp.float32(scale)

    def zrow(r, _):
        def zg(g, _):
            rows_a[r, pl.ds(g * LANES, LANES)] = jnp.zeros((LANES,), jnp.float32)
            return 0
        lax.fori_loop(0, FG, zg, 0)
        return 0
    lax.fori_loop(0, CW, zrow, 0)
    for i in range(ROWS_PER_TEC // CW):
        pltpu.sync_copy(rows_a, acc_s.at[pl.ds(s * ROWS_PER_TEC + i * CW, CW)])
    plsc.subcore_barrier()

    NW = NC * NS

    def issue_idx(j, cv, wv, rv, sem):
        off = (j * NW + wid) * CW
        pltpu.async_copy(col_hbm.at[pl.ds(off, CW)], cv, sem)
        pltpu.async_copy(wh_hbm.at[pl.ds(off, CW)], wv, sem)
        pltpu.async_copy(row_hbm.at[pl.ds(off, CW)], rv, sem)

    def issue_rw(j, wv, rv, sem):
        off = (j * NW + wid) * CW
        pltpu.async_copy(wh_hbm.at[pl.ds(off, CW)], wv, sem)
        pltpu.async_copy(row_hbm.at[pl.ds(off, CW)], rv, sem)

    def issue_col(j, cv, sem):
        off = (j * NW + wid) * CW
        pltpu.async_copy(col_hbm.at[pl.ds(off, CW)], cv, sem)

    def wait_idx(cv, wv, rv, sem):
        pltpu.make_async_copy(col_hbm.at[pl.ds(0, CW)], cv, sem).wait()
        pltpu.make_async_copy(wh_hbm.at[pl.ds(0, CW)], wv, sem).wait()
        pltpu.make_async_copy(row_hbm.at[pl.ds(0, CW)], rv, sem).wait()

    def issue_gather(rv, buf, sem):
        pltpu.async_copy(src_hbm.at[rv], buf, sem)

    def wait_gather(rv, buf, sem):
        pltpu.make_async_copy(src_hbm.at[rv], buf, sem).wait()

    def issue_scatter(cv, buf, sem):
        pltpu.async_copy(buf, acc_s.at[cv], sem, add=True)

    def wait_scatter(cv, buf, sem):
        pltpu.make_async_copy(buf, acc_s.at[cv], sem).wait()

    def scale_buf(wv, buf):
        def edge(e2, _):
            for u in range(2):
                e = e2 * 2 + u
                ee = jnp.full((LANES,), e, jnp.int32)
                bb = plsc.load_gather(wv, [ee]) * sscale
                for f in range(FG):
                    sl = pl.ds(f * LANES, LANES)
                    buf[e, sl] = buf[e, sl] * bb
            return 0
        lax.fori_loop(0, CW // 2, edge, 0)

    issue_idx(0, col_a, wb_a, row_a, sia)
    issue_idx(1, col_b, wb_b, row_b, sib)
    wait_idx(col_a, wb_a, row_a, sia)
    issue_gather(row_a, rows_a, sga)
    wait_idx(col_b, wb_b, row_b, sib)
    issue_gather(row_b, rows_b, sgb)

    def pair(p, _):
        j0 = p * 2

        wait_gather(row_a, rows_a, sga)
        scale_buf(wb_a, rows_a)
        issue_scatter(col_a, rows_a, ssa)

        @pl.when(j0 + 2 < CH2)
        def _():
            issue_rw(j0 + 2, wb_a, row_a, sia)

        wait_gather(row_b, rows_b, sgb)
        scale_buf(wb_b, rows_b)
        issue_scatter(col_b, rows_b, ssb)

        @pl.when(j0 + 3 < CH2)
        def _():
            issue_rw(j0 + 3, wb_b, row_b, sib)

        wait_scatter(col_a, rows_a, ssa)

        @pl.when(j0 + 2 < CH2)
        def _():
            issue_col(j0 + 2, col_a, sia)
            wait_idx(col_a, wb_a, row_a, sia)
            issue_gather(row_a, rows_a, sga)

        wait_scatter(col_b, rows_b, ssb)

        @pl.when(j0 + 3 < CH2)
        def _():
            issue_col(j0 + 3, col_b, sib)
            wait_idx(col_b, wb_b, row_b, sib)
            issue_gather(row_b, rows_b, sgb)
        return 0
    lax.fori_loop(0, CH2 // 2, pair, 0)

    plsc.subcore_barrier()
    pltpu.sync_copy(acc_s.at[pl.ds(s * ROWS_PER_TEC, ROWS_PER_TEC)],
                    out_hbm.at[c, pl.ds(s * ROWS_PER_TEC, ROWS_PER_TEC)])


def _make_prop(scale):
    @functools.partial(
        pl.kernel,
        out_type=jax.ShapeDtypeStruct((NC, N_PAD, IN), jnp.float32),
        mesh=_mesh(),
        compiler_params=pltpu.CompilerParams(needs_layout_passes=False),
        scratch_types=[
            pltpu.VMEM((CW, IN), jnp.float32),
            pltpu.VMEM((CW, IN), jnp.float32),
            pltpu.VMEM((CW,), jnp.int32),
            pltpu.VMEM((CW,), jnp.int32),
            pltpu.VMEM((CW,), jnp.float32),
            pltpu.VMEM((CW,), jnp.float32),
            pltpu.VMEM((CW,), jnp.int32),
            pltpu.VMEM((CW,), jnp.int32),
            pltpu.VMEM_SHARED((N_PAD, IN), jnp.float32),
            pltpu.SemaphoreType.DMA,
            pltpu.SemaphoreType.DMA,
            pltpu.SemaphoreType.DMA,
            pltpu.SemaphoreType.DMA,
            pltpu.SemaphoreType.DMA,
            pltpu.SemaphoreType.DMA,
        ],
    )
    def prop(src_hbm, col_hbm, wh_hbm, row_hbm, out_hbm,
             rows_a, rows_b, col_a, col_b, wb_a, wb_b, row_a, row_b, acc_s,
             sga, sgb, ssa, ssb, sia, sib):
        _prop_tec(scale, src_hbm, col_hbm, wh_hbm, row_hbm, out_hbm,
                  rows_a, rows_b, col_a, col_b, wb_a, wb_b, row_a, row_b,
                  acc_s, sga, sgb, ssa, ssb, sia, sib)
    return prop


_prop1 = _make_prop(1.0)
_prop2 = _make_prop(2.0)

_RB = 1024


def _comb1_body(p_ref, o_ref):
    o_ref[...] = p_ref[0] + p_ref[1]


def _comb2_body(p_ref, z_ref, o_ref):
    o_ref[...] = p_ref[0] + p_ref[1] - z_ref[...]


_comb1 = pl.pallas_call(
    _comb1_body,
    grid=(N_PAD // _RB,),
    in_specs=[pl.BlockSpec((NC, _RB, IN), lambda i: (0, i, 0))],
    out_specs=pl.BlockSpec((_RB, IN), lambda i: (i, 0)),
    out_shape=jax.ShapeDtypeStruct((N_PAD, IN), jnp.float32),
)

_comb2 = pl.pallas_call(
    _comb2_body,
    grid=(N_PAD // _RB,),
    in_specs=[pl.BlockSpec((NC, _RB, IN), lambda i: (0, i, 0)),
              pl.BlockSpec((_RB, IN), lambda i: (i, 0))],
    out_specs=pl.BlockSpec((_RB, IN), lambda i: (i, 0)),
    out_shape=jax.ShapeDtypeStruct((N_PAD, IN), jnp.float32),
)

_MB = 1000
OUT3 = 300


def _mm_body(x_ref, t1_ref, t2_ref, p3_ref, w_ref, b_ref, o_ref):
    t1 = t1_ref[...]
    t3 = p3_ref[0] + p3_ref[1] - t1
    acc = jnp.dot(x_ref[...], w_ref[0], preferred_element_type=jnp.float32)
    acc = acc + jnp.dot(t1, w_ref[1], preferred_element_type=jnp.float32)
    acc = acc + jnp.dot(t2_ref[...], w_ref[2], preferred_element_type=jnp.float32)
    acc = acc + jnp.dot(t3, w_ref[3], preferred_element_type=jnp.float32)
    o_ref[...] = acc + b_ref[0:1, :]


_mm = pl.pallas_call(
    _mm_body,
    grid=(N // _MB,),
    in_specs=[pl.BlockSpec((_MB, IN), lambda i: (i, 0)),
              pl.BlockSpec((_MB, IN), lambda i: (i, 0)),
              pl.BlockSpec((_MB, IN), lambda i: (i, 0)),
              pl.BlockSpec((NC, _MB, IN), lambda i: (0, i, 0)),
              pl.BlockSpec((4, IN, OUT3), lambda i: (0, 0, 0)),
              pl.BlockSpec((8, OUT3), lambda i: (0, 0))],
    out_specs=pl.BlockSpec((_MB, OUT3), lambda i: (i, 0)),
    out_shape=jax.ShapeDtypeStruct((N, OUT3), jnp.float32),
)


def kernel(x, edge_index, edge_weight, W1, b1, W2, b2, W3, b3):
    row = edge_index[0]
    col = edge_index[1]
    e = row.shape[0]
    pad = E_PAD - e
    ar = jnp.arange(pad, dtype=jnp.int32)
    pad_row = ar % jnp.int32(x.shape[0])
    pad_col = jnp.int32(N) + ar % jnp.int32(N_PAD - N)
    row2 = jnp.concatenate([row, pad_row]).reshape(CHUNKS, CW)
    col2 = jnp.concatenate([col, pad_col]).reshape(CHUNKS, CW)
    w2 = jnp.pad(edge_weight, (0, pad)).reshape(CHUNKS, CW)
    x_p = jnp.pad(x, ((0, N_PAD - x.shape[0]), (0, 0)))

    wh = _norm_call(row2, col2, w2)
    wh1 = wh.reshape(E_PAD)
    col1 = col2.reshape(E_PAD)
    row1 = row2.reshape(E_PAD)

    p1 = _prop1(x_p, col1, wh1, row1)
    tx1 = _comb1(p1)
    p2 = _prop2(tx1, col1, wh1, row1)
    tx2 = _comb2(p2, x_p)
    p3 = _prop2(tx2, col1, wh1, row1)

    zero = jnp.zeros((IN, 100), jnp.float32)
    wc = jnp.stack([
        jnp.concatenate([W1[0], W2[0], W3[0]], axis=1),
        jnp.concatenate([W1[1], W2[1], W3[1]], axis=1),
        jnp.concatenate([zero, W2[2], W3[2]], axis=1),
        jnp.concatenate([zero, zero, W3[3]], axis=1),
    ])
    bc = jnp.tile(jnp.concatenate([b1, b2, b3])[None, :], (8, 1))

    return _mm(x_p, tx1, tx2, p3, wc, bc)


# speedup vs baseline: 16.4707x; 1.2070x over previous
import functools

import jax
import jax.numpy as jnp
from jax import lax
from jax.experimental import pallas as pl
from jax.experimental.pallas import tpu as pltpu
from jax.experimental.pallas import tpu_sc as plsc

NC = 2
NS = 16
NW = NC * NS
LANES = 16
CW = 128

N = 10000
IN = 128
N_PAD = 10240
ROWS_PER_TEC = N_PAD // NS
E = 320000
CHUNKS = 2560
E_PAD = CHUNKS * CW
CH1 = CHUNKS // NS
CH2 = CHUNKS // (NC * NS)
FG = IN // LANES

_mesh = functools.partial(
    plsc.VectorSubcoreMesh, core_axis_name="c", subcore_axis_name="s")


def _rsqrt16(d):
    i = lax.bitcast_convert_type(d, jnp.int32)
    i = jnp.int32(0x5F3759DF) - lax.shift_right_logical(i, 1)
    y = lax.bitcast_convert_type(i, jnp.float32)
    half = d * jnp.float32(0.5)
    for _ in range(3):
        y = y * (jnp.float32(1.5) - half * y * y)
    return y


def _norm_tec(row_hbm, col_hbm, w_hbm, wh_hbm, r_v, c_v, w_v, dis_v, zb_v,
              deg_s, sdma):
    c = lax.axis_index("c")
    s = lax.axis_index("s")
    base = s * CH1
    pltpu.sync_copy(row_hbm.at[pl.ds(base, CH1)], r_v)
    pltpu.sync_copy(col_hbm.at[pl.ds(base, CH1)], c_v)
    pltpu.sync_copy(w_hbm.at[pl.ds(base, CH1)], w_v)

    def zz(i, _):
        zb_v[pl.ds(i * LANES, LANES)] = jnp.zeros((LANES,), jnp.float32)
        return 0
    lax.fori_loop(0, ROWS_PER_TEC // LANES, zz, 0)
    pltpu.sync_copy(zb_v, deg_s.at[pl.ds(s * ROWS_PER_TEC, ROWS_PER_TEC)])

    def mask_chunk(j, _):
        def grp(g, _):
            sl = pl.ds(g * LANES, LANES)
            rr = r_v[j, sl]
            cc = c_v[j, sl]
            w_v[j, sl] = jnp.where(rr == cc, jnp.float32(0.0), w_v[j, sl])
            return 0
        lax.fori_loop(0, FG, grp, 0)
        return 0
    lax.fori_loop(0, CH1, mask_chunk, 0)

    plsc.subcore_barrier()

    def deg_blk(b, _):
        for u in range(8):
            j = b * 8 + u
            pltpu.async_copy(w_v.at[j], deg_s.at[r_v.at[j]], sdma, add=True)
        for u in range(8):
            j = b * 8 + u
            pltpu.make_async_copy(w_v.at[j], deg_s.at[r_v.at[j]], sdma).wait()
        return 0
    lax.fori_loop(0, CH1 // 8, deg_blk, 0)

    plsc.subcore_barrier()

    pltpu.sync_copy(deg_s, dis_v)

    def dis_grp(i, _):
        sl = pl.ds(i * LANES, LANES)
        d = dis_v[sl]
        y = _rsqrt16(d)
        dis_v[sl] = jnp.where(d > jnp.float32(0.0), y, jnp.float32(0.0))
        return 0
    lax.fori_loop(0, N_PAD // LANES, dis_grp, 0)

    def wh_chunk(j, _):
        def grp(g, _):
            sl = pl.ds(g * LANES, LANES)
            rr = r_v[j, sl]
            cc = c_v[j, sl]
            dr = plsc.load_gather(dis_v, [rr])
            dc = plsc.load_gather(dis_v, [cc])
            w_v[j, sl] = -(dr * w_v[j, sl] * dc)
            return 0
        lax.fori_loop(0, FG, grp, 0)
        return 0
    lo = c * CH2
    lax.fori_loop(lo, lo + CH2, wh_chunk, 0)
    pltpu.sync_copy(w_v.at[pl.ds(lo, CH2)], wh_hbm.at[pl.ds(base + lo, CH2)])


@functools.partial(
    pl.kernel,
    out_type=jax.ShapeDtypeStruct((CHUNKS, CW), jnp.float32),
    mesh=_mesh(),
    compiler_params=pltpu.CompilerParams(needs_layout_passes=False),
    scratch_types=[
        pltpu.VMEM((CH1, CW), jnp.int32),
        pltpu.VMEM((CH1, CW), jnp.int32),
        pltpu.VMEM((CH1, CW), jnp.float32),
        pltpu.VMEM((N_PAD,), jnp.float32),
        pltpu.VMEM((ROWS_PER_TEC,), jnp.float32),
        pltpu.VMEM_SHARED((N_PAD,), jnp.float32),
        pltpu.SemaphoreType.DMA,
    ],
)
def _norm_call(row_hbm, col_hbm, w_hbm, wh_hbm, r_v, c_v, w_v, dis_v, zb_v,
               deg_s, sdma):
    _norm_tec(row_hbm, col_hbm, w_hbm, wh_hbm, r_v, c_v, w_v, dis_v, zb_v,
              deg_s, sdma)


CWS = 64
CH3 = E_PAD // (NW * CWS)
RING = 4


def _prop_tec(scale, src_hbm, col_hbm, wh_hbm, row_hbm, out_hbm,
              bufs, cols, whs, rws, acc_s, sg, ss, sic, srw):
    c = lax.axis_index("c")
    s = lax.axis_index("s")
    wid = c * NS + s
    sscale = jnp.float32(scale)

    def zrow(r, _):
        def zg(g, _):
            bufs[0][r, pl.ds(g * LANES, LANES)] = jnp.zeros((LANES,), jnp.float32)
            return 0
        lax.fori_loop(0, FG, zg, 0)
        return 0
    lax.fori_loop(0, CWS, zrow, 0)
    for i in range(ROWS_PER_TEC // CWS):
        pltpu.sync_copy(bufs[0], acc_s.at[pl.ds(s * ROWS_PER_TEC + i * CWS, CWS)])
    plsc.subcore_barrier()

    def off(j):
        return (j * NW + wid) * CWS

    def issue_col(j, cv, sem):
        pltpu.async_copy(col_hbm.at[pl.ds(off(j), CWS)], cv, sem)

    def wait_col(cv, sem):
        pltpu.make_async_copy(col_hbm.at[pl.ds(0, CWS)], cv, sem).wait()

    def issue_rw(j, wv, rv, sem):
        pltpu.async_copy(wh_hbm.at[pl.ds(off(j), CWS)], wv, sem)
        pltpu.async_copy(row_hbm.at[pl.ds(off(j), CWS)], rv, sem)

    def wait_rw(wv, rv, sem):
        pltpu.make_async_copy(wh_hbm.at[pl.ds(0, CWS)], wv, sem).wait()
        pltpu.make_async_copy(row_hbm.at[pl.ds(0, CWS)], rv, sem).wait()

    def issue_gather(rv, buf, sem):
        pltpu.async_copy(src_hbm.at[rv], buf, sem)

    def wait_gather(rv, buf, sem):
        pltpu.make_async_copy(src_hbm.at[rv], buf, sem).wait()

    def issue_scatter(cv, buf, sem):
        pltpu.async_copy(buf, acc_s.at[cv], sem, add=True)

    def wait_scatter(cv, buf, sem):
        pltpu.make_async_copy(buf, acc_s.at[cv], sem).wait()

    def scale_buf(wv, buf):
        def edge(e2, _):
            for u in range(2):
                e = e2 * 2 + u
                ee = jnp.full((LANES,), e, jnp.int32)
                bb = plsc.load_gather(wv, [ee]) * sscale
                for f in range(FG):
                    sl = pl.ds(f * LANES, LANES)
                    buf[e, sl] = buf[e, sl] * bb
            return 0
        lax.fori_loop(0, CWS // 2, edge, 0)

    for j0 in range(4):
        issue_col(j0, cols[j0], sic[j0])
    for j0 in range(3):
        issue_rw(j0, whs[j0], rws[j0], srw[j0])
    for j0 in range(2):
        wait_rw(whs[j0], rws[j0], srw[j0])
        wait_col(cols[j0], sic[j0])
        issue_gather(rws[j0], bufs[j0], sg[j0])

    def block(p, _):
        for u in range(8):
            j = p * 8 + u
            b = u % 4
            bn = (u + 2) % 4
            cn = (u + 4) % 8
            c2 = (u + 2) % 8
            rn = (u + 3) % 4

            wait_gather(rws[b], bufs[b], sg[b])
            scale_buf(whs[b], bufs[b])
            issue_scatter(cols[u], bufs[b], ss[b])

            @pl.when(j + 3 < CH3)
            def _():
                issue_rw(j + 3, whs[rn], rws[rn], srw[rn])

            @pl.when(j >= 2)
            def _():
                wait_scatter(cols[(u + 6) % 8], bufs[bn], ss[bn])

            @pl.when(j + 4 < CH3)
            def _():
                issue_col(j + 4, cols[cn], sic[cn])

            @pl.when(j + 2 < CH3)
            def _():
                wait_rw(whs[bn], rws[bn], srw[bn])
                wait_col(cols[c2], sic[c2])
                issue_gather(rws[bn], bufs[bn], sg[bn])
        return 0
    lax.fori_loop(0, CH3 // 8, block, 0)

    wait_scatter(cols[6], bufs[2], ss[2])
    wait_scatter(cols[7], bufs[3], ss[3])

    plsc.subcore_barrier()
    pltpu.sync_copy(acc_s.at[pl.ds(s * ROWS_PER_TEC, ROWS_PER_TEC)],
                    out_hbm.at[c, pl.ds(s * ROWS_PER_TEC, ROWS_PER_TEC)])


def _make_prop(scale):
    scratch = (
        [pltpu.VMEM((CWS, IN), jnp.float32)] * RING
        + [pltpu.VMEM((CWS,), jnp.int32)] * 8
        + [pltpu.VMEM((CWS,), jnp.float32)] * RING
        + [pltpu.VMEM((CWS,), jnp.int32)] * RING
        + [pltpu.VMEM_SHARED((N_PAD, IN), jnp.float32)]
        + [pltpu.SemaphoreType.DMA] * (RING + RING + 8 + RING)
    )

    @functools.partial(
        pl.kernel,
        out_type=jax.ShapeDtypeStruct((NC, N_PAD, IN), jnp.float32),
        mesh=_mesh(),
        compiler_params=pltpu.CompilerParams(needs_layout_passes=False),
        scratch_types=scratch,
    )
    def prop(src_hbm, col_hbm, wh_hbm, row_hbm, out_hbm, *scr):
        bufs = list(scr[0:4])
        cols = list(scr[4:12])
        whs = list(scr[12:16])
        rws = list(scr[16:20])
        acc_s = scr[20]
        sems = list(scr[21:])
        sg = sems[0:4]
        ss = sems[4:8]
        sic = sems[8:16]
        srw = sems[16:20]
        _prop_tec(scale, src_hbm, col_hbm, wh_hbm, row_hbm, out_hbm,
                  bufs, cols, whs, rws, acc_s, sg, ss, sic, srw)
    return prop


_prop1 = _make_prop(1.0)
_prop2 = _make_prop(2.0)

_RB = 1024


def _comb1_body(p_ref, o_ref):
    o_ref[...] = p_ref[0] + p_ref[1]


def _comb2_body(p_ref, z_ref, o_ref):
    o_ref[...] = p_ref[0] + p_ref[1] - z_ref[...]


_comb1 = pl.pallas_call(
    _comb1_body,
    grid=(N_PAD // _RB,),
    in_specs=[pl.BlockSpec((NC, _RB, IN), lambda i: (0, i, 0))],
    out_specs=pl.BlockSpec((_RB, IN), lambda i: (i, 0)),
    out_shape=jax.ShapeDtypeStruct((N_PAD, IN), jnp.float32),
)

_comb2 = pl.pallas_call(
    _comb2_body,
    grid=(N_PAD // _RB,),
    in_specs=[pl.BlockSpec((NC, _RB, IN), lambda i: (0, i, 0)),
              pl.BlockSpec((_RB, IN), lambda i: (i, 0))],
    out_specs=pl.BlockSpec((_RB, IN), lambda i: (i, 0)),
    out_shape=jax.ShapeDtypeStruct((N_PAD, IN), jnp.float32),
)

_MB = 1000
OUT3 = 300


def _mm_body(x_ref, t1_ref, t2_ref, p3_ref, w_ref, b_ref, o_ref):
    t1 = t1_ref[...]
    t3 = p3_ref[0] + p3_ref[1] - t1
    acc = jnp.dot(x_ref[...], w_ref[0], preferred_element_type=jnp.float32)
    acc = acc + jnp.dot(t1, w_ref[1], preferred_element_type=jnp.float32)
    acc = acc + jnp.dot(t2_ref[...], w_ref[2], preferred_element_type=jnp.float32)
    acc = acc + jnp.dot(t3, w_ref[3], preferred_element_type=jnp.float32)
    o_ref[...] = acc + b_ref[0:1, :]


_mm = pl.pallas_call(
    _mm_body,
    grid=(N // _MB,),
    in_specs=[pl.BlockSpec((_MB, IN), lambda i: (i, 0)),
              pl.BlockSpec((_MB, IN), lambda i: (i, 0)),
              pl.BlockSpec((_MB, IN), lambda i: (i, 0)),
              pl.BlockSpec((NC, _MB, IN), lambda i: (0, i, 0)),
              pl.BlockSpec((4, IN, OUT3), lambda i: (0, 0, 0)),
              pl.BlockSpec((8, OUT3), lambda i: (0, 0))],
    out_specs=pl.BlockSpec((_MB, OUT3), lambda i: (i, 0)),
    out_shape=jax.ShapeDtypeStruct((N, OUT3), jnp.float32),
)


def kernel(x, edge_index, edge_weight, W1, b1, W2, b2, W3, b3):
    row = edge_index[0]
    col = edge_index[1]
    e = row.shape[0]
    pad = E_PAD - e
    ar = jnp.arange(pad, dtype=jnp.int32)
    pad_row = ar % jnp.int32(x.shape[0])
    pad_col = jnp.int32(N) + ar % jnp.int32(N_PAD - N)
    row2 = jnp.concatenate([row, pad_row]).reshape(CHUNKS, CW)
    col2 = jnp.concatenate([col, pad_col]).reshape(CHUNKS, CW)
    w2 = jnp.pad(edge_weight, (0, pad)).reshape(CHUNKS, CW)
    x_p = jnp.pad(x, ((0, N_PAD - x.shape[0]), (0, 0)))

    wh = _norm_call(row2, col2, w2)
    wh1 = wh.reshape(E_PAD)
    col1 = col2.reshape(E_PAD)
    row1 = row2.reshape(E_PAD)

    p1 = _prop1(x_p, col1, wh1, row1)
    tx1 = _comb1(p1)
    p2 = _prop2(tx1, col1, wh1, row1)
    tx2 = _comb2(p2, x_p)
    p3 = _prop2(tx2, col1, wh1, row1)

    zero = jnp.zeros((IN, 100), jnp.float32)
    wc = jnp.stack([
        jnp.concatenate([W1[0], W2[0], W3[0]], axis=1),
        jnp.concatenate([W1[1], W2[1], W3[1]], axis=1),
        jnp.concatenate([zero, W2[2], W3[2]], axis=1),
        jnp.concatenate([zero, zero, W3[3]], axis=1),
    ])
    bc = jnp.tile(jnp.concatenate([b1, b2, b3])[None, :], (8, 1))

    return _mm(x_p, tx1, tx2, p3, wc, bc)


# speedup vs baseline: 19.2024x; 1.1659x over previous
import functools

import jax
import jax.numpy as jnp
from jax import lax
from jax.experimental import pallas as pl
from jax.experimental.pallas import tpu as pltpu
from jax.experimental.pallas import tpu_sc as plsc

NC = 2
NS = 16
NW = NC * NS
LANES = 16
CW = 128

N = 10000
IN = 128
N_PAD = 10240
ROWS_PER_TEC = N_PAD // NS
E = 320000
CHUNKS = 2560
E_PAD = CHUNKS * CW
CH1 = CHUNKS // NS
CH2 = CHUNKS // (NC * NS)
FG = IN // LANES

_mesh = functools.partial(
    plsc.VectorSubcoreMesh, core_axis_name="c", subcore_axis_name="s")


def _rsqrt16(d):
    i = lax.bitcast_convert_type(d, jnp.int32)
    i = jnp.int32(0x5F3759DF) - lax.shift_right_logical(i, 1)
    y = lax.bitcast_convert_type(i, jnp.float32)
    half = d * jnp.float32(0.5)
    for _ in range(3):
        y = y * (jnp.float32(1.5) - half * y * y)
    return y


def _norm_tec(row_hbm, col_hbm, w_hbm, wh_hbm, r_v, c_v, w_v, dis_v, zb_v,
              deg_s, sdma):
    c = lax.axis_index("c")
    s = lax.axis_index("s")
    base = s * CH1
    pltpu.sync_copy(row_hbm.at[pl.ds(base, CH1)], r_v)
    pltpu.sync_copy(col_hbm.at[pl.ds(base, CH1)], c_v)
    pltpu.sync_copy(w_hbm.at[pl.ds(base, CH1)], w_v)

    def zz(i, _):
        zb_v[pl.ds(i * LANES, LANES)] = jnp.zeros((LANES,), jnp.float32)
        return 0
    lax.fori_loop(0, ROWS_PER_TEC // LANES, zz, 0)
    pltpu.sync_copy(zb_v, deg_s.at[pl.ds(s * ROWS_PER_TEC, ROWS_PER_TEC)])

    def mask_chunk(j, _):
        def grp(g, _):
            sl = pl.ds(g * LANES, LANES)
            rr = r_v[j, sl]
            cc = c_v[j, sl]
            w_v[j, sl] = jnp.where(rr == cc, jnp.float32(0.0), w_v[j, sl])
            return 0
        lax.fori_loop(0, FG, grp, 0)
        return 0
    lax.fori_loop(0, CH1, mask_chunk, 0)

    plsc.subcore_barrier()

    def deg_blk(b, _):
        for u in range(8):
            j = b * 8 + u
            pltpu.async_copy(w_v.at[j], deg_s.at[r_v.at[j]], sdma, add=True)
        for u in range(8):
            j = b * 8 + u
            pltpu.make_async_copy(w_v.at[j], deg_s.at[r_v.at[j]], sdma).wait()
        return 0
    lax.fori_loop(0, CH1 // 8, deg_blk, 0)

    plsc.subcore_barrier()

    pltpu.sync_copy(deg_s, dis_v)

    def dis_grp(i, _):
        sl = pl.ds(i * LANES, LANES)
        d = dis_v[sl]
        y = _rsqrt16(d)
        dis_v[sl] = jnp.where(d > jnp.float32(0.0), y, jnp.float32(0.0))
        return 0
    lax.fori_loop(0, N_PAD // LANES, dis_grp, 0)

    def wh_chunk(j, _):
        def grp(g, _):
            sl = pl.ds(g * LANES, LANES)
            rr = r_v[j, sl]
            cc = c_v[j, sl]
            dr = plsc.load_gather(dis_v, [rr])
            dc = plsc.load_gather(dis_v, [cc])
            w_v[j, sl] = -(dr * w_v[j, sl] * dc)
            return 0
        lax.fori_loop(0, FG, grp, 0)
        return 0
    lo = c * CH2
    lax.fori_loop(lo, lo + CH2, wh_chunk, 0)
    pltpu.sync_copy(w_v.at[pl.ds(lo, CH2)], wh_hbm.at[pl.ds(base + lo, CH2)])


@functools.partial(
    pl.kernel,
    out_type=jax.ShapeDtypeStruct((CHUNKS, CW), jnp.float32),
    mesh=_mesh(),
    compiler_params=pltpu.CompilerParams(needs_layout_passes=False),
    scratch_types=[
        pltpu.VMEM((CH1, CW), jnp.int32),
        pltpu.VMEM((CH1, CW), jnp.int32),
        pltpu.VMEM((CH1, CW), jnp.float32),
        pltpu.VMEM((N_PAD,), jnp.float32),
        pltpu.VMEM((ROWS_PER_TEC,), jnp.float32),
        pltpu.VMEM_SHARED((N_PAD,), jnp.float32),
        pltpu.SemaphoreType.DMA,
    ],
)
def _norm_call(row_hbm, col_hbm, w_hbm, wh_hbm, r_v, c_v, w_v, dis_v, zb_v,
               deg_s, sdma):
    _norm_tec(row_hbm, col_hbm, w_hbm, wh_hbm, r_v, c_v, w_v, dis_v, zb_v,
              deg_s, sdma)


CWS = 64
CH3 = E_PAD // (NW * CWS)
RING = 5
NCOL = 2 * RING


def _prop_tec(scale, src_hbm, col_hbm, wh_hbm, row_hbm, out_hbm,
              bufs, cols, whs, rws, acc_s, sg, ss, sic, srw):
    c = lax.axis_index("c")
    s = lax.axis_index("s")
    wid = c * NS + s
    sscale = jnp.float32(scale)

    def zrow(r, _):
        def zg(g, _):
            bufs[0][r, pl.ds(g * LANES, LANES)] = jnp.zeros((LANES,), jnp.float32)
            return 0
        lax.fori_loop(0, FG, zg, 0)
        return 0
    lax.fori_loop(0, CWS, zrow, 0)
    for i in range(ROWS_PER_TEC // CWS):
        pltpu.sync_copy(bufs[0], acc_s.at[pl.ds(s * ROWS_PER_TEC + i * CWS, CWS)])
    plsc.subcore_barrier()

    def off(j):
        return (j * NW + wid) * CWS

    def issue_col(j, cv, sem):
        pltpu.async_copy(col_hbm.at[pl.ds(off(j), CWS)], cv, sem)

    def wait_col(cv, sem):
        pltpu.make_async_copy(col_hbm.at[pl.ds(0, CWS)], cv, sem).wait()

    def issue_rw(j, wv, rv, sem):
        pltpu.async_copy(wh_hbm.at[pl.ds(off(j), CWS)], wv, sem)
        pltpu.async_copy(row_hbm.at[pl.ds(off(j), CWS)], rv, sem)

    def wait_rw(wv, rv, sem):
        pltpu.make_async_copy(wh_hbm.at[pl.ds(0, CWS)], wv, sem).wait()
        pltpu.make_async_copy(row_hbm.at[pl.ds(0, CWS)], rv, sem).wait()

    def issue_gather(rv, buf, sem):
        pltpu.async_copy(src_hbm.at[rv], buf, sem)

    def wait_gather(rv, buf, sem):
        pltpu.make_async_copy(src_hbm.at[rv], buf, sem).wait()

    def issue_scatter(cv, buf, sem):
        pltpu.async_copy(buf, acc_s.at[cv], sem, add=True)

    def wait_scatter(cv, buf, sem):
        pltpu.make_async_copy(buf, acc_s.at[cv], sem).wait()

    def scale_buf(wv, buf):
        def edge(e2, _):
            for u in range(2):
                e = e2 * 2 + u
                ee = jnp.full((LANES,), e, jnp.int32)
                bb = plsc.load_gather(wv, [ee]) * sscale
                for f in range(FG):
                    sl = pl.ds(f * LANES, LANES)
                    buf[e, sl] = buf[e, sl] * bb
            return 0
        lax.fori_loop(0, CWS // 2, edge, 0)

    for j0 in range(RING):
        issue_col(j0, cols[j0], sic[j0])
    for j0 in range(RING - 1):
        issue_rw(j0, whs[j0], rws[j0], srw[j0])
    for j0 in range(RING - 2):
        wait_rw(whs[j0], rws[j0], srw[j0])
        wait_col(cols[j0], sic[j0])
        issue_gather(rws[j0], bufs[j0], sg[j0])

    def block(p, _):
        for u in range(NCOL):
            j = p * NCOL + u
            b = u % RING
            b3 = (u + 3) % RING
            rn = (u + 4) % RING
            ccur = u
            cnew = (u + RING) % NCOL
            c3 = (u + 3) % NCOL

            wait_gather(rws[b], bufs[b], sg[b])
            scale_buf(whs[b], bufs[b])
            issue_scatter(cols[ccur], bufs[b], ss[b])

            @pl.when(j + 4 < CH3)
            def _():
                issue_rw(j + 4, whs[rn], rws[rn], srw[rn])

            @pl.when(j >= 2)
            def _():
                wait_scatter(cols[(u + 8) % NCOL], bufs[b3], ss[b3])

            @pl.when(j + RING < CH3)
            def _():
                issue_col(j + RING, cols[cnew], sic[cnew])

            @pl.when(j + 3 < CH3)
            def _():
                wait_rw(whs[b3], rws[b3], srw[b3])
                wait_col(cols[c3], sic[c3])
                issue_gather(rws[b3], bufs[b3], sg[b3])
        return 0
    lax.fori_loop(0, CH3 // NCOL, block, 0)

    wait_scatter(cols[8], bufs[3], ss[3])
    wait_scatter(cols[9], bufs[4], ss[4])

    plsc.subcore_barrier()
    pltpu.sync_copy(acc_s.at[pl.ds(s * ROWS_PER_TEC, ROWS_PER_TEC)],
                    out_hbm.at[c, pl.ds(s * ROWS_PER_TEC, ROWS_PER_TEC)])


def _make_prop(scale):
    scratch = (
        [pltpu.VMEM((CWS, IN), jnp.float32)] * RING
        + [pltpu.VMEM((CWS,), jnp.int32)] * NCOL
        + [pltpu.VMEM((CWS,), jnp.float32)] * RING
        + [pltpu.VMEM((CWS,), jnp.int32)] * RING
        + [pltpu.VMEM_SHARED((N_PAD, IN), jnp.float32)]
        + [pltpu.SemaphoreType.DMA] * (RING + RING + NCOL + RING)
    )

    @functools.partial(
        pl.kernel,
        out_type=jax.ShapeDtypeStruct((NC, N_PAD, IN), jnp.float32),
        mesh=_mesh(),
        compiler_params=pltpu.CompilerParams(needs_layout_passes=False),
        scratch_types=scratch,
    )
    def prop(src_hbm, col_hbm, wh_hbm, row_hbm, out_hbm, *scr):
        bufs = list(scr[0:RING])
        cols = list(scr[RING:RING + NCOL])
        whs = list(scr[RING + NCOL:2 * RING + NCOL])
        rws = list(scr[2 * RING + NCOL:3 * RING + NCOL])
        acc_s = scr[3 * RING + NCOL]
        sems = list(scr[3 * RING + NCOL + 1:])
        sg = sems[0:RING]
        ss = sems[RING:2 * RING]
        sic = sems[2 * RING:2 * RING + NCOL]
        srw = sems[2 * RING + NCOL:]
        _prop_tec(scale, src_hbm, col_hbm, wh_hbm, row_hbm, out_hbm,
                  bufs, cols, whs, rws, acc_s, sg, ss, sic, srw)
    return prop


_prop1 = _make_prop(1.0)
_prop2 = _make_prop(2.0)

_RB = 1024


def _comb1_body(p_ref, o_ref):
    o_ref[...] = p_ref[0] + p_ref[1]


def _comb2_body(p_ref, z_ref, o_ref):
    o_ref[...] = p_ref[0] + p_ref[1] - z_ref[...]


_comb1 = pl.pallas_call(
    _comb1_body,
    grid=(N_PAD // _RB,),
    in_specs=[pl.BlockSpec((NC, _RB, IN), lambda i: (0, i, 0))],
    out_specs=pl.BlockSpec((_RB, IN), lambda i: (i, 0)),
    out_shape=jax.ShapeDtypeStruct((N_PAD, IN), jnp.float32),
)

_comb2 = pl.pallas_call(
    _comb2_body,
    grid=(N_PAD // _RB,),
    in_specs=[pl.BlockSpec((NC, _RB, IN), lambda i: (0, i, 0)),
              pl.BlockSpec((_RB, IN), lambda i: (i, 0))],
    out_specs=pl.BlockSpec((_RB, IN), lambda i: (i, 0)),
    out_shape=jax.ShapeDtypeStruct((N_PAD, IN), jnp.float32),
)

_MB = 1000
OUT3 = 300


def _mm_body(x_ref, t1_ref, t2_ref, p3_ref, w_ref, b_ref, o_ref):
    t1 = t1_ref[...]
    t3 = p3_ref[0] + p3_ref[1] - t1
    acc = jnp.dot(x_ref[...], w_ref[0], preferred_element_type=jnp.float32)
    acc = acc + jnp.dot(t1, w_ref[1], preferred_element_type=jnp.float32)
    acc = acc + jnp.dot(t2_ref[...], w_ref[2], preferred_element_type=jnp.float32)
    acc = acc + jnp.dot(t3, w_ref[3], preferred_element_type=jnp.float32)
    o_ref[...] = acc + b_ref[0:1, :]


_mm = pl.pallas_call(
    _mm_body,
    grid=(N // _MB,),
    in_specs=[pl.BlockSpec((_MB, IN), lambda i: (i, 0)),
              pl.BlockSpec((_MB, IN), lambda i: (i, 0)),
              pl.BlockSpec((_MB, IN), lambda i: (i, 0)),
              pl.BlockSpec((NC, _MB, IN), lambda i: (0, i, 0)),
              pl.BlockSpec((4, IN, OUT3), lambda i: (0, 0, 0)),
              pl.BlockSpec((8, OUT3), lambda i: (0, 0))],
    out_specs=pl.BlockSpec((_MB, OUT3), lambda i: (i, 0)),
    out_shape=jax.ShapeDtypeStruct((N, OUT3), jnp.float32),
)


def kernel(x, edge_index, edge_weight, W1, b1, W2, b2, W3, b3):
    row = edge_index[0]
    col = edge_index[1]
    e = row.shape[0]
    pad = E_PAD - e
    ar = jnp.arange(pad, dtype=jnp.int32)
    pad_row = ar % jnp.int32(x.shape[0])
    pad_col = jnp.int32(N) + ar % jnp.int32(N_PAD - N)
    row2 = jnp.concatenate([row, pad_row]).reshape(CHUNKS, CW)
    col2 = jnp.concatenate([col, pad_col]).reshape(CHUNKS, CW)
    w2 = jnp.pad(edge_weight, (0, pad)).reshape(CHUNKS, CW)
    x_p = jnp.pad(x, ((0, N_PAD - x.shape[0]), (0, 0)))

    wh = _norm_call(row2, col2, w2)
    wh1 = wh.reshape(E_PAD)
    col1 = col2.reshape(E_PAD)
    row1 = row2.reshape(E_PAD)

    p1 = _prop1(x_p, col1, wh1, row1)
    tx1 = _comb1(p1)
    p2 = _prop2(tx1, col1, wh1, row1)
    tx2 = _comb2(p2, x_p)
    p3 = _prop2(tx2, col1, wh1, row1)

    zero = jnp.zeros((IN, 100), jnp.float32)
    wc = jnp.stack([
        jnp.concatenate([W1[0], W2[0], W3[0]], axis=1),
        jnp.concatenate([W1[1], W2[1], W3[1]], axis=1),
        jnp.concatenate([zero, W2[2], W3[2]], axis=1),
        jnp.concatenate([zero, zero, W3[3]], axis=1),
    ])
    bc = jnp.tile(jnp.concatenate([b1, b2, b3])[None, :], (8, 1))

    return _mm(x_p, tx1, tx2, p3, wc, bc)


# speedup vs baseline: 19.8447x; 1.0334x over previous
import functools

import jax
import jax.numpy as jnp
from jax import lax
from jax.experimental import pallas as pl
from jax.experimental.pallas import tpu as pltpu
from jax.experimental.pallas import tpu_sc as plsc

NC = 2
NS = 16
NW = NC * NS
LANES = 16
CW = 128

N = 10000
IN = 128
N_PAD = 10240
ROWS_PER_TEC = N_PAD // NS
E = 320000
CHUNKS = 2560
E_PAD = CHUNKS * CW
CH1 = CHUNKS // NS
CH2 = CHUNKS // (NC * NS)
FG = IN // LANES

_mesh = functools.partial(
    plsc.VectorSubcoreMesh, core_axis_name="c", subcore_axis_name="s")


def _rsqrt16(d):
    i = lax.bitcast_convert_type(d, jnp.int32)
    i = jnp.int32(0x5F3759DF) - lax.shift_right_logical(i, 1)
    y = lax.bitcast_convert_type(i, jnp.float32)
    half = d * jnp.float32(0.5)
    for _ in range(3):
        y = y * (jnp.float32(1.5) - half * y * y)
    return y


def _norm_tec(row_hbm, col_hbm, w_hbm, wh_hbm, r_v, c_v, w_v, dis_v, zb_v,
              deg_s, sdma, sdmb):
    c = lax.axis_index("c")
    s = lax.axis_index("s")
    base = s * CH1
    pltpu.async_copy(row_hbm.at[pl.ds(base, CH1)], r_v, sdma)
    pltpu.async_copy(col_hbm.at[pl.ds(base, CH1)], c_v, sdma)
    pltpu.async_copy(w_hbm.at[pl.ds(base, CH1)], w_v, sdma)
    pltpu.make_async_copy(row_hbm.at[pl.ds(base, CH1)], r_v, sdma).wait()
    pltpu.make_async_copy(col_hbm.at[pl.ds(base, CH1)], c_v, sdma).wait()
    pltpu.make_async_copy(w_hbm.at[pl.ds(base, CH1)], w_v, sdma).wait()

    def zz(i, _):
        zb_v[pl.ds(i * LANES, LANES)] = jnp.zeros((LANES,), jnp.float32)
        return 0
    lax.fori_loop(0, ROWS_PER_TEC // LANES, zz, 0)
    pltpu.sync_copy(zb_v, deg_s.at[pl.ds(s * ROWS_PER_TEC, ROWS_PER_TEC)])

    def mask_chunk(j, _):
        def grp(g, _):
            sl = pl.ds(g * LANES, LANES)
            rr = r_v[j, sl]
            cc = c_v[j, sl]
            w_v[j, sl] = jnp.where(rr == cc, jnp.float32(0.0), w_v[j, sl])
            return 0
        lax.fori_loop(0, FG, grp, 0)
        return 0
    lax.fori_loop(0, CH1, mask_chunk, 0)

    plsc.subcore_barrier()

    def deg_blk(b2, _):
        for half, sem in ((0, sdma), (1, sdmb)):
            b = b2 * 2 + half
            for u in range(8):
                j = b * 8 + u
                pltpu.async_copy(w_v.at[j], deg_s.at[r_v.at[j]], sem, add=True)
        for half, sem in ((0, sdma), (1, sdmb)):
            b = b2 * 2 + half
            for u in range(8):
                j = b * 8 + u
                pltpu.make_async_copy(w_v.at[j], deg_s.at[r_v.at[j]], sem).wait()
        return 0
    lax.fori_loop(0, CH1 // 16, deg_blk, 0)

    plsc.subcore_barrier()

    pltpu.sync_copy(deg_s, dis_v)

    def dis_grp(i, _):
        sl = pl.ds(i * LANES, LANES)
        d = dis_v[sl]
        y = _rsqrt16(d)
        dis_v[sl] = jnp.where(d > jnp.float32(0.0), y, jnp.float32(0.0))
        return 0
    lax.fori_loop(0, N_PAD // LANES, dis_grp, 0)

    def wh_chunk(j, _):
        def grp(g, _):
            sl = pl.ds(g * LANES, LANES)
            rr = r_v[j, sl]
            cc = c_v[j, sl]
            dr = plsc.load_gather(dis_v, [rr])
            dc = plsc.load_gather(dis_v, [cc])
            w_v[j, sl] = -(dr * w_v[j, sl] * dc)
            return 0
        lax.fori_loop(0, FG, grp, 0)
        return 0
    lo = c * CH2
    lax.fori_loop(lo, lo + CH2, wh_chunk, 0)
    pltpu.sync_copy(w_v.at[pl.ds(lo, CH2)], wh_hbm.at[pl.ds(base + lo, CH2)])


@functools.partial(
    pl.kernel,
    out_type=jax.ShapeDtypeStruct((CHUNKS, CW), jnp.float32),
    mesh=_mesh(),
    compiler_params=pltpu.CompilerParams(needs_layout_passes=False),
    scratch_types=[
        pltpu.VMEM((CH1, CW), jnp.int32),
        pltpu.VMEM((CH1, CW), jnp.int32),
        pltpu.VMEM((CH1, CW), jnp.float32),
        pltpu.VMEM((N_PAD,), jnp.float32),
        pltpu.VMEM((ROWS_PER_TEC,), jnp.float32),
        pltpu.VMEM_SHARED((N_PAD,), jnp.float32),
        pltpu.SemaphoreType.DMA,
        pltpu.SemaphoreType.DMA,
    ],
)
def _norm_call(row_hbm, col_hbm, w_hbm, wh_hbm, r_v, c_v, w_v, dis_v, zb_v,
               deg_s, sdma, sdmb):
    _norm_tec(row_hbm, col_hbm, w_hbm, wh_hbm, r_v, c_v, w_v, dis_v, zb_v,
              deg_s, sdma, sdmb)


CWS = 64
CH3 = E_PAD // (NW * CWS)
RING = 5
NCOL = 2 * RING


def _prop_tec(scale, src_hbm, col_hbm, wh_hbm, row_hbm, out_hbm,
              bufs, cols, whs, rws, acc_s, sg, ss, sic, srw):
    c = lax.axis_index("c")
    s = lax.axis_index("s")
    wid = c * NS + s
    del scale

    def zrow(r, _):
        def zg(g, _):
            bufs[0][r, pl.ds(g * LANES, LANES)] = jnp.zeros((LANES,), jnp.float32)
            return 0
        lax.fori_loop(0, FG, zg, 0)
        return 0
    lax.fori_loop(0, CWS, zrow, 0)
    for i in range(ROWS_PER_TEC // CWS):
        pltpu.sync_copy(bufs[0], acc_s.at[pl.ds(s * ROWS_PER_TEC + i * CWS, CWS)])
    plsc.subcore_barrier()

    def off(j):
        return (j * NW + wid) * CWS

    def issue_col(j, cv, sem):
        pltpu.async_copy(col_hbm.at[pl.ds(off(j), CWS)], cv, sem)

    def wait_col(cv, sem):
        pltpu.make_async_copy(col_hbm.at[pl.ds(0, CWS)], cv, sem).wait()

    def issue_rw(j, wv, rv, sem):
        pltpu.async_copy(wh_hbm.at[pl.ds(off(j), CWS)], wv, sem)
        pltpu.async_copy(row_hbm.at[pl.ds(off(j), CWS)], rv, sem)

    def wait_rw(wv, rv, sem):
        pltpu.make_async_copy(wh_hbm.at[pl.ds(0, CWS)], wv, sem).wait()
        pltpu.make_async_copy(row_hbm.at[pl.ds(0, CWS)], rv, sem).wait()

    def issue_gather(rv, buf, sem):
        pltpu.async_copy(src_hbm.at[rv], buf, sem)

    def wait_gather(rv, buf, sem):
        pltpu.make_async_copy(src_hbm.at[rv], buf, sem).wait()

    def issue_scatter(cv, buf, sem):
        pltpu.async_copy(buf, acc_s.at[cv], sem, add=True)

    def wait_scatter(cv, buf, sem):
        pltpu.make_async_copy(buf, acc_s.at[cv], sem).wait()

    def scale_buf(wv, buf):
        def edge(e2, _):
            for u in range(2):
                e = e2 * 2 + u
                ee = jnp.full((LANES,), e, jnp.int32)
                bb = plsc.load_gather(wv, [ee])
                for f in range(FG):
                    sl = pl.ds(f * LANES, LANES)
                    buf[e, sl] = buf[e, sl] * bb
            return 0
        lax.fori_loop(0, CWS // 2, edge, 0)

    for j0 in range(RING):
        issue_col(j0, cols[j0], sic[j0])
    for j0 in range(RING - 1):
        issue_rw(j0, whs[j0], rws[j0], srw[j0])
    for j0 in range(RING - 2):
        wait_rw(whs[j0], rws[j0], srw[j0])
        wait_col(cols[j0], sic[j0])
        issue_gather(rws[j0], bufs[j0], sg[j0])

    def block(p, _):
        for u in range(NCOL):
            j = p * NCOL + u
            b = u % RING
            b3 = (u + 3) % RING
            rn = (u + 4) % RING
            ccur = u
            cnew = (u + RING) % NCOL
            c3 = (u + 3) % NCOL

            wait_gather(rws[b], bufs[b], sg[b])
            scale_buf(whs[b], bufs[b])
            issue_scatter(cols[ccur], bufs[b], ss[b])

            @pl.when(j + 4 < CH3)
            def _():
                issue_rw(j + 4, whs[rn], rws[rn], srw[rn])

            @pl.when(j >= 2)
            def _():
                wait_scatter(cols[(u + 8) % NCOL], bufs[b3], ss[b3])

            @pl.when(j + RING < CH3)
            def _():
                issue_col(j + RING, cols[cnew], sic[cnew])

            @pl.when(j + 3 < CH3)
            def _():
                wait_rw(whs[b3], rws[b3], srw[b3])
                wait_col(cols[c3], sic[c3])
                issue_gather(rws[b3], bufs[b3], sg[b3])
        return 0
    lax.fori_loop(0, CH3 // NCOL, block, 0)

    wait_scatter(cols[8], bufs[3], ss[3])
    wait_scatter(cols[9], bufs[4], ss[4])

    plsc.subcore_barrier()
    pltpu.sync_copy(acc_s.at[pl.ds(s * ROWS_PER_TEC, ROWS_PER_TEC)],
                    out_hbm.at[c, pl.ds(s * ROWS_PER_TEC, ROWS_PER_TEC)])


def _make_prop(scale):
    scratch = (
        [pltpu.VMEM((CWS, IN), jnp.float32)] * RING
        + [pltpu.VMEM((CWS,), jnp.int32)] * NCOL
        + [pltpu.VMEM((CWS,), jnp.float32)] * RING
        + [pltpu.VMEM((CWS,), jnp.int32)] * RING
        + [pltpu.VMEM_SHARED((N_PAD, IN), jnp.float32)]
        + [pltpu.SemaphoreType.DMA] * (RING + RING + NCOL + RING)
    )

    @functools.partial(
        pl.kernel,
        out_type=jax.ShapeDtypeStruct((NC, N_PAD, IN), jnp.float32),
        mesh=_mesh(),
        compiler_params=pltpu.CompilerParams(needs_layout_passes=False),
        scratch_types=scratch,
    )
    def prop(src_hbm, col_hbm, wh_hbm, row_hbm, out_hbm, *scr):
        bufs = list(scr[0:RING])
        cols = list(scr[RING:RING + NCOL])
        whs = list(scr[RING + NCOL:2 * RING + NCOL])
        rws = list(scr[2 * RING + NCOL:3 * RING + NCOL])
        acc_s = scr[3 * RING + NCOL]
        sems = list(scr[3 * RING + NCOL + 1:])
        sg = sems[0:RING]
        ss = sems[RING:2 * RING]
        sic = sems[2 * RING:2 * RING + NCOL]
        srw = sems[2 * RING + NCOL:]
        _prop_tec(scale, src_hbm, col_hbm, wh_hbm, row_hbm, out_hbm,
                  bufs, cols, whs, rws, acc_s, sg, ss, sic, srw)
    return prop


_prop = _make_prop(None)

_RB = 1024


def _comb1_body(p_ref, o_ref):
    o_ref[...] = p_ref[0] + p_ref[1]


def _comb2_body(p_ref, z_ref, o_ref):
    o_ref[...] = (p_ref[0] + p_ref[1]) * jnp.float32(2.0) - z_ref[...]


_comb1 = pl.pallas_call(
    _comb1_body,
    grid=(N_PAD // _RB,),
    in_specs=[pl.BlockSpec((NC, _RB, IN), lambda i: (0, i, 0))],
    out_specs=pl.BlockSpec((_RB, IN), lambda i: (i, 0)),
    out_shape=jax.ShapeDtypeStruct((N_PAD, IN), jnp.float32),
)

_comb2 = pl.pallas_call(
    _comb2_body,
    grid=(N_PAD // _RB,),
    in_specs=[pl.BlockSpec((NC, _RB, IN), lambda i: (0, i, 0)),
              pl.BlockSpec((_RB, IN), lambda i: (i, 0))],
    out_specs=pl.BlockSpec((_RB, IN), lambda i: (i, 0)),
    out_shape=jax.ShapeDtypeStruct((N_PAD, IN), jnp.float32),
)

_MB = 1000
OUT3 = 300


def _mm_body(x_ref, t1_ref, t2_ref, p3_ref, w_ref, b_ref, o_ref):
    t1 = t1_ref[...]
    t3 = (p3_ref[0] + p3_ref[1]) * jnp.float32(2.0) - t1
    acc = jnp.dot(x_ref[...], w_ref[0], preferred_element_type=jnp.float32)
    acc = acc + jnp.dot(t1, w_ref[1], preferred_element_type=jnp.float32)
    acc = acc + jnp.dot(t2_ref[...], w_ref[2], preferred_element_type=jnp.float32)
    acc = acc + jnp.dot(t3, w_ref[3], preferred_element_type=jnp.float32)
    o_ref[...] = acc + b_ref[0:1, :]


_mm = pl.pallas_call(
    _mm_body,
    grid=(N // _MB,),
    in_specs=[pl.BlockSpec((_MB, IN), lambda i: (i, 0)),
              pl.BlockSpec((_MB, IN), lambda i: (i, 0)),
              pl.BlockSpec((_MB, IN), lambda i: (i, 0)),
              pl.BlockSpec((NC, _MB, IN), lambda i: (0, i, 0)),
              pl.BlockSpec((4, IN, OUT3), lambda i: (0, 0, 0)),
              pl.BlockSpec((8, OUT3), lambda i: (0, 0))],
    out_specs=pl.BlockSpec((_MB, OUT3), lambda i: (i, 0)),
    out_shape=jax.ShapeDtypeStruct((N, OUT3), jnp.float32),
)


def kernel(x, edge_index, edge_weight, W1, b1, W2, b2, W3, b3):
    row = edge_index[0]
    col = edge_index[1]
    e = row.shape[0]
    pad = E_PAD - e
    ar = jnp.arange(pad, dtype=jnp.int32)
    pad_row = ar % jnp.int32(x.shape[0])
    pad_col = jnp.int32(N) + ar % jnp.int32(N_PAD - N)
    row2 = jnp.concatenate([row, pad_row]).reshape(CHUNKS, CW)
    col2 = jnp.concatenate([col, pad_col]).reshape(CHUNKS, CW)
    w2 = jnp.pad(edge_weight, (0, pad)).reshape(CHUNKS, CW)
    x_p = jnp.pad(x, ((0, N_PAD - x.shape[0]), (0, 0)))

    wh = _norm_call(row2, col2, w2)
    wh1 = wh.reshape(E_PAD)
    col1 = col2.reshape(E_PAD)
    row1 = row2.reshape(E_PAD)

    p1 = _prop(x_p, col1, wh1, row1)
    tx1 = _comb1(p1)
    p2 = _prop(tx1, col1, wh1, row1)
    tx2 = _comb2(p2, x_p)
    p3 = _prop(tx2, col1, wh1, row1)

    zero = jnp.zeros((IN, 100), jnp.float32)
    wc = jnp.stack([
        jnp.concatenate([W1[0], W2[0], W3[0]], axis=1),
        jnp.concatenate([W1[1], W2[1], W3[1]], axis=1),
        jnp.concatenate([zero, W2[2], W3[2]], axis=1),
        jnp.concatenate([zero, zero, W3[3]], axis=1),
    ])
    bc = jnp.tile(jnp.concatenate([b1, b2, b3])[None, :], (8, 1))

    return _mm(x_p, tx1, tx2, p3, wc, bc)
